# Initial kernel scaffold; baseline (speedup 1.0000x reference)
import jax
import jax.numpy as jnp
from jax.experimental import pallas as pl


def kernel(x, edge_index, edge_index_2, W1, a_src1, a_dst1, b1, W2, a_src2, a_dst2, b2, W3, a_src3, a_dst3, b3):
    raise NotImplementedError("write your pallas kernel here")


# speedup vs baseline: 15.6581x; 15.6581x over previous
import functools

import jax
import jax.numpy as jnp
from jax import lax
from jax.experimental import pallas as pl
from jax.experimental.pallas import tpu as pltpu
from jax.experimental.pallas import tpu_sc as plsc

N = 10000
E = 320000
EP = E + N
F_IN = 128
HID = 64
HEADS = 8
OUT = 64
AUG = 128

NTILES = 16
B = 128
PER_TILE = 20736
EPAD = PER_TILE * NTILES
NPAD = 10240
RPT = NPAD // NTILES
NB = 10
BN = N // NB

_f32 = jnp.float32
_i32 = jnp.int32


def _tc_pre1_body(x_ref, w_ref, asr_ref, adr_ref,
                  haug_ref, als_ref, ald_ref, mxs_ref, mxd_ref):
    i = pl.program_id(0)

    @pl.when(i == 0)
    def _():
        mxs_ref[...] = jnp.full((HEADS, 128), -1e30, _f32)
        mxd_ref[...] = jnp.full((HEADS, 128), -1e30, _f32)

    h = jnp.dot(x_ref[...], w_ref[...], preferred_element_type=_f32)
    ones = jnp.ones((BN, 1), _f32)
    zpad = jnp.zeros((BN, AUG - HID - 1), _f32)
    a_s_all, a_d_all = [], []
    for hd in range(HEADS):
        hh = h[:, hd * HID:(hd + 1) * HID]
        a_s = jnp.sum(hh * asr_ref[hd, :][None, :], axis=1)
        a_d = jnp.sum(hh * adr_ref[hd, :][None, :], axis=1)
        a_s_all.append(a_s)
        a_d_all.append(a_d)
        mxs_ref[hd, :] = jnp.maximum(mxs_ref[hd, :], jnp.max(a_s))
        mxd_ref[hd, :] = jnp.maximum(mxd_ref[hd, :], jnp.max(a_d))
        haug_ref[hd, :, :] = jnp.concatenate([hh, ones, zpad], axis=1)
    als_ref[...] = jnp.stack(a_s_all, axis=1)
    ald_ref[...] = jnp.stack(a_d_all, axis=1)


def _tc_pre1(x, W1, a_src1, a_dst1):
    return pl.pallas_call(
        _tc_pre1_body,
        grid=(NB,),
        in_specs=[
            pl.BlockSpec((BN, F_IN), lambda i: (i, 0)),
            pl.BlockSpec((F_IN, HEADS * HID), lambda i: (0, 0)),
            pl.BlockSpec((HEADS, HID), lambda i: (0, 0)),
            pl.BlockSpec((HEADS, HID), lambda i: (0, 0)),
        ],
        out_specs=[
            pl.BlockSpec((HEADS, BN, AUG), lambda i: (0, i, 0)),
            pl.BlockSpec((BN, HEADS), lambda i: (i, 0)),
            pl.BlockSpec((BN, HEADS), lambda i: (i, 0)),
            pl.BlockSpec((HEADS, 128), lambda i: (0, 0)),
            pl.BlockSpec((HEADS, 128), lambda i: (0, 0)),
        ],
        out_shape=[
            jax.ShapeDtypeStruct((HEADS, N, AUG), _f32),
            jax.ShapeDtypeStruct((N, HEADS), _f32),
            jax.ShapeDtypeStruct((N, HEADS), _f32),
            jax.ShapeDtypeStruct((HEADS, 128), _f32),
            jax.ShapeDtypeStruct((HEADS, 128), _f32),
        ],
    )(x, W1, a_src1, a_dst1)


def _sc_chunk_loop(haug, sidx, didx, acc, tab_s, tab_d, svec, dvec, gidx,
                   rows, eev, bnd_v, sem, ebase0, goff):
    def chunk(i, _):
        eb = ebase0 + i * B
        pltpu.sync_copy(sidx.at[pl.ds(eb, B)], svec)
        pltpu.sync_copy(didx.at[pl.ds(eb, B)], dvec)

        def sub16(k, _):
            o = k * 16
            sid = svec[pl.ds(o, 16)]
            did = dvec[pl.ds(o, 16)]
            gidx[pl.ds(o, 16)] = sid + goff
            e = plsc.load_gather(tab_s, [sid]) + plsc.load_gather(tab_d, [did])
            e = jnp.where(e > 0, e, 0.2 * e)
            eev[pl.ds(o, 16)] = jnp.exp(e - bnd_v[...])
            return 0

        lax.fori_loop(0, B // 16, sub16, 0)
        pltpu.async_copy(haug.at[gidx], rows, sem).wait()

        def scale(j, _):
            ej = plsc.load_gather(eev, [jnp.broadcast_to(j, (16,)).astype(_i32)])
            for q in range(AUG // 16):
                rows[j, pl.ds(q * 16, 16)] = rows[j, pl.ds(q * 16, 16)] * ej
            return 0

        lax.fori_loop(0, B, scale, 0)
        pltpu.sync_copy(rows, acc.at[dvec], add=True)
        return 0

    lax.fori_loop(0, PER_TILE // B, chunk, 0)


def _sc_l1_body(haug, sidx, didx, alps, alpd, mxs, mxd, zrs, out,
                tab_s, tab_d, svec, dvec, gidx, rows, eev,
                m1v, m2v, bnd_v, acc, sem):
    core = lax.axis_index("c")
    sub = lax.axis_index("s")
    ebase0 = sub * PER_TILE
    rbase = sub * RPT
    for hl in range(HEADS // 2):
        head = core * (HEADS // 2) + hl
        pltpu.sync_copy(zrs, acc.at[pl.ds(rbase, RPT)])
        pltpu.sync_copy(alps.at[pl.ds(head * NPAD, NPAD)], tab_s)
        pltpu.sync_copy(alpd.at[pl.ds(head * NPAD, NPAD)], tab_d)
        pltpu.sync_copy(mxs.at[pl.ds(head * 128, 16)], m1v)
        pltpu.sync_copy(mxd.at[pl.ds(head * 128, 16)], m2v)
        b = m1v[...] + m2v[...]
        bnd_v[...] = jnp.where(b > 0, b, 0.2 * b)
        plsc.subcore_barrier()
        _sc_chunk_loop(haug, sidx, didx, acc, tab_s, tab_d, svec, dvec,
                       gidx, rows, eev, bnd_v, sem, ebase0, head * N)
        plsc.subcore_barrier()
        pltpu.sync_copy(acc.at[pl.ds(rbase, RPT)],
                        out.at[pl.ds(head * NPAD + rbase, RPT)])
        plsc.subcore_barrier()


def _sc_l1(haug_f, sidx, didx, alps_f, alpd_f, mxs_f, mxd_f, zrs):
    mesh = plsc.VectorSubcoreMesh(core_axis_name="c", subcore_axis_name="s")
    return pl.kernel(
        _sc_l1_body,
        out_type=jax.ShapeDtypeStruct((HEADS * NPAD, AUG), _f32),
        mesh=mesh,
        compiler_params=pltpu.CompilerParams(needs_layout_passes=False),
        scratch_types=[
            pltpu.VMEM((NPAD,), _f32),
            pltpu.VMEM((NPAD,), _f32),
            pltpu.VMEM((B,), _i32),
            pltpu.VMEM((B,), _i32),
            pltpu.VMEM((B,), _i32),
            pltpu.VMEM((B, AUG), _f32),
            pltpu.VMEM((B,), _f32),
            pltpu.VMEM((16,), _f32),
            pltpu.VMEM((16,), _f32),
            pltpu.VMEM((16,), _f32),
            pltpu.VMEM_SHARED((NPAD, AUG), _f32),
            pltpu.SemaphoreType.DMA,
        ],
    )(haug_f, sidx, didx, alps_f, alpd_f, mxs_f, mxd_f, zrs)


def _sc_l23_body(haug, sidx, didx, alps, mxs, zrs, out,
                 tab_s, tab_d, svec, dvec, gidx, rows, eev,
                 m1v, m2v, bnd_v, acc, sem):
    core = lax.axis_index("c")
    sub = lax.axis_index("s")
    ebase0 = core * EPAD + sub * PER_TILE
    rbase = sub * RPT
    row_s = 2 * core
    row_d = 2 * core + 1
    pltpu.sync_copy(zrs, acc.at[pl.ds(rbase, RPT)])
    pltpu.sync_copy(alps.at[pl.ds(row_s * NPAD, NPAD)], tab_s)
    pltpu.sync_copy(alps.at[pl.ds(row_d * NPAD, NPAD)], tab_d)
    pltpu.sync_copy(mxs.at[pl.ds(row_s * 128, 16)], m1v)
    pltpu.sync_copy(mxs.at[pl.ds(row_d * 128, 16)], m2v)
    b = m1v[...] + m2v[...]
    bnd_v[...] = jnp.where(b > 0, b, 0.2 * b)
    plsc.subcore_barrier()
    _sc_chunk_loop(haug, sidx, didx, acc, tab_s, tab_d, svec, dvec,
                   gidx, rows, eev, bnd_v, sem, ebase0, core * N)
    plsc.subcore_barrier()
    pltpu.sync_copy(acc.at[pl.ds(rbase, RPT)],
                    out.at[pl.ds(core * NPAD + rbase, RPT)])


def _sc_l23(haug_f, sidx2, didx2, alps_f, mxs_f, zrs):
    mesh = plsc.VectorSubcoreMesh(core_axis_name="c", subcore_axis_name="s")
    return pl.kernel(
        _sc_l23_body,
        out_type=jax.ShapeDtypeStruct((2 * NPAD, AUG), _f32),
        mesh=mesh,
        compiler_params=pltpu.CompilerParams(needs_layout_passes=False),
        scratch_types=[
            pltpu.VMEM((NPAD,), _f32),
            pltpu.VMEM((NPAD,), _f32),
            pltpu.VMEM((B,), _i32),
            pltpu.VMEM((B,), _i32),
            pltpu.VMEM((B,), _i32),
            pltpu.VMEM((B, AUG), _f32),
            pltpu.VMEM((B,), _f32),
            pltpu.VMEM((16,), _f32),
            pltpu.VMEM((16,), _f32),
            pltpu.VMEM((16,), _f32),
            pltpu.VMEM_SHARED((NPAD, AUG), _f32),
            pltpu.SemaphoreType.DMA,
        ],
    )(haug_f, sidx2, didx2, alps_f, mxs_f, zrs)


def _tc_mid_body(acc_ref, b1_ref, w2_ref, as2_ref, ad2_ref,
                 w3_ref, as3_ref, ad3_ref,
                 haug2_ref, haug3_ref, al_ref, mx_ref):
    i = pl.program_id(0)

    @pl.when(i == 0)
    def _():
        mx_ref[...] = jnp.full((HEADS, 128), -1e30, _f32)

    cols = []
    for hd in range(HEADS):
        num = acc_ref[hd, :, 0:HID]
        den = acc_ref[hd, :, HID:HID + 1]
        cols.append(num / (den + 1e-16))
    emb = jnp.concatenate(cols, axis=1) + b1_ref[0, :][None, :]
    emb = jnp.where(emb > 0, emb, jnp.exp(jnp.minimum(emb, 0.0)) - 1.0)

    ones = jnp.ones((BN, 1), _f32)
    zpad = jnp.zeros((BN, AUG - OUT - 1), _f32)

    h2 = jnp.dot(emb, w2_ref[...], preferred_element_type=_f32)
    a_s2 = jnp.sum(h2 * as2_ref[0, :][None, :], axis=1)
    a_d2 = jnp.sum(h2 * ad2_ref[0, :][None, :], axis=1)
    haug2_ref[...] = jnp.concatenate([h2, ones, zpad], axis=1)

    h3 = jnp.dot(emb, w3_ref[...], preferred_element_type=_f32)
    a_s3 = jnp.sum(h3 * as3_ref[0, :][None, :], axis=1)
    a_d3 = jnp.sum(h3 * ad3_ref[0, :][None, :], axis=1)
    haug3_ref[...] = jnp.concatenate([h3, ones, zpad], axis=1)

    zcol = jnp.zeros((BN,), _f32)
    al_ref[...] = jnp.stack(
        [a_s2, a_d2, a_s3, a_d3] + [zcol] * (HEADS - 4), axis=1)
    mx_ref[0, :] = jnp.maximum(mx_ref[0, :], jnp.max(a_s2))
    mx_ref[1, :] = jnp.maximum(mx_ref[1, :], jnp.max(a_d2))
    mx_ref[2, :] = jnp.maximum(mx_ref[2, :], jnp.max(a_s3))
    mx_ref[3, :] = jnp.maximum(mx_ref[3, :], jnp.max(a_d3))


def _tc_mid(acc1, b1, W2, a_src2, a_dst2, W3, a_src3, a_dst3):
    return pl.pallas_call(
        _tc_mid_body,
        grid=(NB,),
        in_specs=[
            pl.BlockSpec((HEADS, BN, AUG), lambda i: (0, i, 0)),
            pl.BlockSpec((1, HEADS * HID), lambda i: (0, 0)),
            pl.BlockSpec((HEADS * HID, OUT), lambda i: (0, 0)),
            pl.BlockSpec((1, OUT), lambda i: (0, 0)),
            pl.BlockSpec((1, OUT), lambda i: (0, 0)),
            pl.BlockSpec((HEADS * HID, OUT), lambda i: (0, 0)),
            pl.BlockSpec((1, OUT), lambda i: (0, 0)),
            pl.BlockSpec((1, OUT), lambda i: (0, 0)),
        ],
        out_specs=[
            pl.BlockSpec((BN, AUG), lambda i: (i, 0)),
            pl.BlockSpec((BN, AUG), lambda i: (i, 0)),
            pl.BlockSpec((BN, HEADS), lambda i: (i, 0)),
            pl.BlockSpec((HEADS, 128), lambda i: (0, 0)),
        ],
        out_shape=[
            jax.ShapeDtypeStruct((N, AUG), _f32),
            jax.ShapeDtypeStruct((N, AUG), _f32),
            jax.ShapeDtypeStruct((N, HEADS), _f32),
            jax.ShapeDtypeStruct((HEADS, 128), _f32),
        ],
    )(acc1, b1, W2, a_src2, a_dst2, W3, a_src3, a_dst3)


def _tc_post_body(acc2_ref, acc3_ref, b2_ref, b3_ref,
                  lg1_ref, lg2_ref, prd_ref):
    x1 = acc2_ref[:, 0:OUT] / (acc2_ref[:, HID:HID + 1] + 1e-16) \
        + b2_ref[0, :][None, :]
    x2 = acc3_ref[:, 0:OUT] / (acc3_ref[:, HID:HID + 1] + 1e-16) \
        + b3_ref[0, :][None, :]
    m1 = jnp.max(x1, axis=1, keepdims=True)
    p1 = jnp.exp(x1 - m1)
    lg1_ref[...] = p1 / jnp.sum(p1, axis=1, keepdims=True)
    m2 = jnp.max(x2, axis=1, keepdims=True)
    p2 = jnp.exp(x2 - m2)
    lg2_ref[...] = p2 / jnp.sum(p2, axis=1, keepdims=True)
    ii = lax.broadcasted_iota(_i32, (BN, OUT), 1)
    cand = jnp.where(x1 == m1, ii, OUT)
    prd_ref[0, 0, :] = jnp.min(cand, axis=1)


def _tc_post(acc2, acc3, b2, b3):
    return pl.pallas_call(
        _tc_post_body,
        grid=(NB,),
        in_specs=[
            pl.BlockSpec((BN, AUG), lambda i: (i, 0)),
            pl.BlockSpec((BN, AUG), lambda i: (i, 0)),
            pl.BlockSpec((1, OUT), lambda i: (0, 0)),
            pl.BlockSpec((1, OUT), lambda i: (0, 0)),
        ],
        out_specs=[
            pl.BlockSpec((BN, OUT), lambda i: (i, 0)),
            pl.BlockSpec((BN, OUT), lambda i: (i, 0)),
            pl.BlockSpec((1, 1, BN), lambda i: (i, 0, 0)),
        ],
        out_shape=[
            jax.ShapeDtypeStruct((N, OUT), _f32),
            jax.ShapeDtypeStruct((N, OUT), _f32),
            jax.ShapeDtypeStruct((NB, 1, BN), _i32),
        ],
    )(acc2, acc3, b2, b3)


def _pad_edges(src, dst):
    loop = jnp.arange(N, dtype=_i32)
    npad = EPAD - EP
    s = jnp.concatenate([src.astype(_i32), loop,
                         jnp.zeros((npad,), _i32)])
    d = jnp.concatenate([dst.astype(_i32), loop,
                         jnp.full((npad,), N, _i32)])
    return s, d


def kernel(x, edge_index, edge_index_2, W1, a_src1, a_dst1, b1,
           W2, a_src2, a_dst2, b2, W3, a_src3, a_dst3, b3):
    s1, d1 = _pad_edges(edge_index[0], edge_index[1])
    s2, d2 = _pad_edges(edge_index_2[0], edge_index_2[1])
    zrs = jnp.zeros((RPT, AUG), _f32)

    haug1, als1, ald1, mxs1, mxd1 = _tc_pre1(x, W1, a_src1, a_dst1)
    haug1_f = haug1.reshape(HEADS * N, AUG)
    als1_f = jnp.pad(als1.T, ((0, 0), (0, NPAD - N))).reshape(-1)
    ald1_f = jnp.pad(ald1.T, ((0, 0), (0, NPAD - N))).reshape(-1)

    acc1 = _sc_l1(haug1_f, s1, d1, als1_f, ald1_f,
                  mxs1.reshape(-1), mxd1.reshape(-1), zrs)
    acc1 = acc1.reshape(HEADS, NPAD, AUG)

    haug2, haug3, al23, mx23 = _tc_mid(
        acc1, b1.reshape(1, -1), W2, a_src2, a_dst2, W3, a_src3, a_dst3)
    haug23_f = jnp.concatenate([haug2, haug3], axis=0)
    al23_f = jnp.pad(al23.T, ((0, 0), (0, NPAD - N))).reshape(-1)
    s23 = jnp.concatenate([s1, s2])
    d23 = jnp.concatenate([d1, d2])

    acc23 = _sc_l23(haug23_f, s23, d23, al23_f, mx23.reshape(-1), zrs)

    logits, logits2, preds = _tc_post(acc23[:NPAD], acc23[NPAD:],
                                      b2.reshape(1, -1), b3.reshape(1, -1))
    return (logits, logits2, preds.reshape(-1))


# speedup vs baseline: 19.3059x; 1.2330x over previous
import functools

import jax
import jax.numpy as jnp
from jax import lax
from jax.experimental import pallas as pl
from jax.experimental.pallas import tpu as pltpu
from jax.experimental.pallas import tpu_sc as plsc

N = 10000
E = 320000
EP = E + N
F_IN = 128
HID = 64
HEADS = 8
OUT = 64
AUG = 128

NTILES = 16
B = 64
PER_TILE = 20736
EPAD = PER_TILE * NTILES
NPAD = 10240
RPT = NPAD // NTILES
NB = 10
BN = N // NB

_f32 = jnp.float32
_i32 = jnp.int32


def _tc_pre1_body(x_ref, w_ref, asr_ref, adr_ref,
                  haug_ref, als_ref, ald_ref, mxs_ref, mxd_ref):
    i = pl.program_id(0)

    @pl.when(i == 0)
    def _():
        mxs_ref[...] = jnp.full((HEADS, 128), -1e30, _f32)
        mxd_ref[...] = jnp.full((HEADS, 128), -1e30, _f32)

    h = jnp.dot(x_ref[...], w_ref[...], preferred_element_type=_f32)
    ones = jnp.ones((BN, 1), _f32)
    zpad = jnp.zeros((BN, AUG - HID - 1), _f32)
    a_s_all, a_d_all = [], []
    for hd in range(HEADS):
        hh = h[:, hd * HID:(hd + 1) * HID]
        a_s = jnp.sum(hh * asr_ref[hd, :][None, :], axis=1)
        a_d = jnp.sum(hh * adr_ref[hd, :][None, :], axis=1)
        a_s_all.append(a_s)
        a_d_all.append(a_d)
        mxs_ref[hd, :] = jnp.maximum(mxs_ref[hd, :], jnp.max(a_s))
        mxd_ref[hd, :] = jnp.maximum(mxd_ref[hd, :], jnp.max(a_d))
        haug_ref[hd, :, :] = jnp.concatenate([hh, ones, zpad], axis=1)
    als_ref[...] = jnp.stack(a_s_all, axis=1)
    ald_ref[...] = jnp.stack(a_d_all, axis=1)


def _tc_pre1(x, W1, a_src1, a_dst1):
    return pl.pallas_call(
        _tc_pre1_body,
        grid=(NB,),
        in_specs=[
            pl.BlockSpec((BN, F_IN), lambda i: (i, 0)),
            pl.BlockSpec((F_IN, HEADS * HID), lambda i: (0, 0)),
            pl.BlockSpec((HEADS, HID), lambda i: (0, 0)),
            pl.BlockSpec((HEADS, HID), lambda i: (0, 0)),
        ],
        out_specs=[
            pl.BlockSpec((HEADS, BN, AUG), lambda i: (0, i, 0)),
            pl.BlockSpec((BN, HEADS), lambda i: (i, 0)),
            pl.BlockSpec((BN, HEADS), lambda i: (i, 0)),
            pl.BlockSpec((HEADS, 128), lambda i: (0, 0)),
            pl.BlockSpec((HEADS, 128), lambda i: (0, 0)),
        ],
        out_shape=[
            jax.ShapeDtypeStruct((HEADS, N, AUG), _f32),
            jax.ShapeDtypeStruct((N, HEADS), _f32),
            jax.ShapeDtypeStruct((N, HEADS), _f32),
            jax.ShapeDtypeStruct((HEADS, 128), _f32),
            jax.ShapeDtypeStruct((HEADS, 128), _f32),
        ],
    )(x, W1, a_src1, a_dst1)


NCH = PER_TILE // B
NSC = (HID + 16) // 16


def _sc_chunk_loop(haug, sidx, didx, acc, tab_s, tab_d, svec, dvec, gidx,
                   rows, eev, gsem, ssem, bnd_v, ebase0, goff):
    def prep(c, b):
        eb = ebase0 + c * B
        pltpu.sync_copy(sidx.at[pl.ds(eb, B)], svec[b])
        pltpu.sync_copy(didx.at[pl.ds(eb, B)], dvec[b])

        def sub16(k, _):
            o = k * 16
            sid = svec[b][pl.ds(o, 16)]
            did = dvec[b][pl.ds(o, 16)]
            gidx[b][pl.ds(o, 16)] = sid + goff
            e = (plsc.load_gather(tab_s, [sid])
                 + plsc.load_gather(tab_d, [did]))
            e = jnp.where(e > 0, e, 0.2 * e)
            eev[b][pl.ds(o, 16)] = jnp.exp(e - bnd_v[...])
            return 0

        lax.fori_loop(0, B // 16, sub16, 0, unroll=2)
        pltpu.async_copy(haug.at[gidx[b]], rows[b], gsem[b])

    def drain(b):
        pltpu.make_async_copy(rows[b], acc.at[dvec[b]], ssem[b]).wait()

    def finish(b):
        pltpu.make_async_copy(haug.at[gidx[b]], rows[b], gsem[b]).wait()

        def scale(j, _):
            ej = plsc.load_gather(
                eev[b], [jnp.broadcast_to(j, (16,)).astype(_i32)])
            for q in range(NSC):
                rows[b][j, pl.ds(q * 16, 16)] = \
                    rows[b][j, pl.ds(q * 16, 16)] * ej
            return 0

        lax.fori_loop(0, B, scale, 0, unroll=2)
        pltpu.async_copy(rows[b], acc.at[dvec[b]], ssem[b], add=True)

    prep(0, 0)

    def outer(g, _):
        for b in range(2):
            c = 2 * g + b

            @pl.when(c + 1 < NCH)
            def _():
                @pl.when(c >= 1)
                def _():
                    drain(1 - b)
                prep(c + 1, 1 - b)

            finish(b)
        return 0

    lax.fori_loop(0, NCH // 2, outer, 0)
    drain(0)
    drain(1)


def _sc_l1_body(haug, sidx, didx, alps, alpd, mxs, mxd, zrs, out,
                tab_s, tab_d, svec0, svec1, dvec0, dvec1, gidx0, gidx1,
                rows0, rows1, eev0, eev1,
                m1v, m2v, bnd_v, acc, gsem0, gsem1, ssem0, ssem1):
    core = lax.axis_index("c")
    sub = lax.axis_index("s")
    ebase0 = sub * PER_TILE
    rbase = sub * RPT
    for hl in range(HEADS // 2):
        head = core * (HEADS // 2) + hl
        pltpu.sync_copy(zrs, acc.at[pl.ds(rbase, RPT)])
        pltpu.sync_copy(alps.at[pl.ds(head * NPAD, NPAD)], tab_s)
        pltpu.sync_copy(alpd.at[pl.ds(head * NPAD, NPAD)], tab_d)
        pltpu.sync_copy(mxs.at[pl.ds(head * 128, 16)], m1v)
        pltpu.sync_copy(mxd.at[pl.ds(head * 128, 16)], m2v)
        b = m1v[...] + m2v[...]
        bnd_v[...] = jnp.where(b > 0, b, 0.2 * b)
        plsc.subcore_barrier()
        _sc_chunk_loop(haug, sidx, didx, acc, tab_s, tab_d,
                       (svec0, svec1), (dvec0, dvec1), (gidx0, gidx1),
                       (rows0, rows1), (eev0, eev1),
                       (gsem0, gsem1), (ssem0, ssem1),
                       bnd_v, ebase0, head * N)
        plsc.subcore_barrier()
        pltpu.sync_copy(acc.at[pl.ds(rbase, RPT)],
                        out.at[pl.ds(head * NPAD + rbase, RPT)])
        plsc.subcore_barrier()


def _sc_l1(haug_f, sidx, didx, alps_f, alpd_f, mxs_f, mxd_f, zrs):
    mesh = plsc.VectorSubcoreMesh(core_axis_name="c", subcore_axis_name="s")
    return pl.kernel(
        _sc_l1_body,
        out_type=jax.ShapeDtypeStruct((HEADS * NPAD, AUG), _f32),
        mesh=mesh,
        compiler_params=pltpu.CompilerParams(needs_layout_passes=False),
        scratch_types=_sc_scratch(),
    )(haug_f, sidx, didx, alps_f, alpd_f, mxs_f, mxd_f, zrs)


def _sc_scratch():
    return [
        pltpu.VMEM((NPAD,), _f32),
        pltpu.VMEM((NPAD,), _f32),
        pltpu.VMEM((B,), _i32),
        pltpu.VMEM((B,), _i32),
        pltpu.VMEM((B,), _i32),
        pltpu.VMEM((B,), _i32),
        pltpu.VMEM((B,), _i32),
        pltpu.VMEM((B,), _i32),
        pltpu.VMEM((B, AUG), _f32),
        pltpu.VMEM((B, AUG), _f32),
        pltpu.VMEM((B,), _f32),
        pltpu.VMEM((B,), _f32),
        pltpu.VMEM((16,), _f32),
        pltpu.VMEM((16,), _f32),
        pltpu.VMEM((16,), _f32),
        pltpu.VMEM_SHARED((NPAD, AUG), _f32),
        pltpu.SemaphoreType.DMA,
        pltpu.SemaphoreType.DMA,
        pltpu.SemaphoreType.DMA,
        pltpu.SemaphoreType.DMA,
    ]


def _sc_l23_body(haug, sidx, didx, alps, mxs, zrs, out,
                 tab_s, tab_d, svec0, svec1, dvec0, dvec1, gidx0, gidx1,
                 rows0, rows1, eev0, eev1,
                 m1v, m2v, bnd_v, acc, gsem0, gsem1, ssem0, ssem1):
    core = lax.axis_index("c")
    sub = lax.axis_index("s")
    ebase0 = core * EPAD + sub * PER_TILE
    rbase = sub * RPT
    row_s = 2 * core
    row_d = 2 * core + 1
    pltpu.sync_copy(zrs, acc.at[pl.ds(rbase, RPT)])
    pltpu.sync_copy(alps.at[pl.ds(row_s * NPAD, NPAD)], tab_s)
    pltpu.sync_copy(alps.at[pl.ds(row_d * NPAD, NPAD)], tab_d)
    pltpu.sync_copy(mxs.at[pl.ds(row_s * 128, 16)], m1v)
    pltpu.sync_copy(mxs.at[pl.ds(row_d * 128, 16)], m2v)
    b = m1v[...] + m2v[...]
    bnd_v[...] = jnp.where(b > 0, b, 0.2 * b)
    plsc.subcore_barrier()
    _sc_chunk_loop(haug, sidx, didx, acc, tab_s, tab_d,
                   (svec0, svec1), (dvec0, dvec1), (gidx0, gidx1),
                   (rows0, rows1), (eev0, eev1),
                   (gsem0, gsem1), (ssem0, ssem1),
                   bnd_v, ebase0, core * N)
    plsc.subcore_barrier()
    pltpu.sync_copy(acc.at[pl.ds(rbase, RPT)],
                    out.at[pl.ds(core * NPAD + rbase, RPT)])


def _sc_l23(haug_f, sidx2, didx2, alps_f, mxs_f, zrs):
    mesh = plsc.VectorSubcoreMesh(core_axis_name="c", subcore_axis_name="s")
    return pl.kernel(
        _sc_l23_body,
        out_type=jax.ShapeDtypeStruct((2 * NPAD, AUG), _f32),
        mesh=mesh,
        compiler_params=pltpu.CompilerParams(needs_layout_passes=False),
        scratch_types=_sc_scratch(),
    )(haug_f, sidx2, didx2, alps_f, mxs_f, zrs)


def _tc_mid_body(acc_ref, b1_ref, w2_ref, as2_ref, ad2_ref,
                 w3_ref, as3_ref, ad3_ref,
                 haug2_ref, haug3_ref, al_ref, mx_ref):
    i = pl.program_id(0)

    @pl.when(i == 0)
    def _():
        mx_ref[...] = jnp.full((HEADS, 128), -1e30, _f32)

    cols = []
    for hd in range(HEADS):
        num = acc_ref[hd, :, 0:HID]
        den = acc_ref[hd, :, HID:HID + 1]
        cols.append(num / (den + 1e-16))
    emb = jnp.concatenate(cols, axis=1) + b1_ref[0, :][None, :]
    emb = jnp.where(emb > 0, emb, jnp.exp(jnp.minimum(emb, 0.0)) - 1.0)

    ones = jnp.ones((BN, 1), _f32)
    zpad = jnp.zeros((BN, AUG - OUT - 1), _f32)

    h2 = jnp.dot(emb, w2_ref[...], preferred_element_type=_f32)
    a_s2 = jnp.sum(h2 * as2_ref[0, :][None, :], axis=1)
    a_d2 = jnp.sum(h2 * ad2_ref[0, :][None, :], axis=1)
    haug2_ref[...] = jnp.concatenate([h2, ones, zpad], axis=1)

    h3 = jnp.dot(emb, w3_ref[...], preferred_element_type=_f32)
    a_s3 = jnp.sum(h3 * as3_ref[0, :][None, :], axis=1)
    a_d3 = jnp.sum(h3 * ad3_ref[0, :][None, :], axis=1)
    haug3_ref[...] = jnp.concatenate([h3, ones, zpad], axis=1)

    zcol = jnp.zeros((BN,), _f32)
    al_ref[...] = jnp.stack(
        [a_s2, a_d2, a_s3, a_d3] + [zcol] * (HEADS - 4), axis=1)
    mx_ref[0, :] = jnp.maximum(mx_ref[0, :], jnp.max(a_s2))
    mx_ref[1, :] = jnp.maximum(mx_ref[1, :], jnp.max(a_d2))
    mx_ref[2, :] = jnp.maximum(mx_ref[2, :], jnp.max(a_s3))
    mx_ref[3, :] = jnp.maximum(mx_ref[3, :], jnp.max(a_d3))


def _tc_mid(acc1, b1, W2, a_src2, a_dst2, W3, a_src3, a_dst3):
    return pl.pallas_call(
        _tc_mid_body,
        grid=(NB,),
        in_specs=[
            pl.BlockSpec((HEADS, BN, AUG), lambda i: (0, i, 0)),
            pl.BlockSpec((1, HEADS * HID), lambda i: (0, 0)),
            pl.BlockSpec((HEADS * HID, OUT), lambda i: (0, 0)),
            pl.BlockSpec((1, OUT), lambda i: (0, 0)),
            pl.BlockSpec((1, OUT), lambda i: (0, 0)),
            pl.BlockSpec((HEADS * HID, OUT), lambda i: (0, 0)),
            pl.BlockSpec((1, OUT), lambda i: (0, 0)),
            pl.BlockSpec((1, OUT), lambda i: (0, 0)),
        ],
        out_specs=[
            pl.BlockSpec((BN, AUG), lambda i: (i, 0)),
            pl.BlockSpec((BN, AUG), lambda i: (i, 0)),
            pl.BlockSpec((BN, HEADS), lambda i: (i, 0)),
            pl.BlockSpec((HEADS, 128), lambda i: (0, 0)),
        ],
        out_shape=[
            jax.ShapeDtypeStruct((N, AUG), _f32),
            jax.ShapeDtypeStruct((N, AUG), _f32),
            jax.ShapeDtypeStruct((N, HEADS), _f32),
            jax.ShapeDtypeStruct((HEADS, 128), _f32),
        ],
    )(acc1, b1, W2, a_src2, a_dst2, W3, a_src3, a_dst3)


def _tc_post_body(acc2_ref, acc3_ref, b2_ref, b3_ref,
                  lg1_ref, lg2_ref, prd_ref):
    x1 = acc2_ref[:, 0:OUT] / (acc2_ref[:, HID:HID + 1] + 1e-16) \
        + b2_ref[0, :][None, :]
    x2 = acc3_ref[:, 0:OUT] / (acc3_ref[:, HID:HID + 1] + 1e-16) \
        + b3_ref[0, :][None, :]
    m1 = jnp.max(x1, axis=1, keepdims=True)
    p1 = jnp.exp(x1 - m1)
    lg1_ref[...] = p1 / jnp.sum(p1, axis=1, keepdims=True)
    m2 = jnp.max(x2, axis=1, keepdims=True)
    p2 = jnp.exp(x2 - m2)
    lg2_ref[...] = p2 / jnp.sum(p2, axis=1, keepdims=True)
    ii = lax.broadcasted_iota(_i32, (BN, OUT), 1)
    cand = jnp.where(x1 == m1, ii, OUT)
    prd_ref[0, 0, :] = jnp.min(cand, axis=1)


def _tc_post(acc2, acc3, b2, b3):
    return pl.pallas_call(
        _tc_post_body,
        grid=(NB,),
        in_specs=[
            pl.BlockSpec((BN, AUG), lambda i: (i, 0)),
            pl.BlockSpec((BN, AUG), lambda i: (i, 0)),
            pl.BlockSpec((1, OUT), lambda i: (0, 0)),
            pl.BlockSpec((1, OUT), lambda i: (0, 0)),
        ],
        out_specs=[
            pl.BlockSpec((BN, OUT), lambda i: (i, 0)),
            pl.BlockSpec((BN, OUT), lambda i: (i, 0)),
            pl.BlockSpec((1, 1, BN), lambda i: (i, 0, 0)),
        ],
        out_shape=[
            jax.ShapeDtypeStruct((N, OUT), _f32),
            jax.ShapeDtypeStruct((N, OUT), _f32),
            jax.ShapeDtypeStruct((NB, 1, BN), _i32),
        ],
    )(acc2, acc3, b2, b3)


def _pad_edges(src, dst):
    loop = jnp.arange(N, dtype=_i32)
    npad = EPAD - EP
    s = jnp.concatenate([src.astype(_i32), loop,
                         jnp.zeros((npad,), _i32)])
    d = jnp.concatenate([dst.astype(_i32), loop,
                         jnp.full((npad,), N, _i32)])
    return s, d


def kernel(x, edge_index, edge_index_2, W1, a_src1, a_dst1, b1,
           W2, a_src2, a_dst2, b2, W3, a_src3, a_dst3, b3):
    s1, d1 = _pad_edges(edge_index[0], edge_index[1])
    s2, d2 = _pad_edges(edge_index_2[0], edge_index_2[1])
    zrs = jnp.zeros((RPT, AUG), _f32)

    haug1, als1, ald1, mxs1, mxd1 = _tc_pre1(x, W1, a_src1, a_dst1)
    haug1_f = haug1.reshape(HEADS * N, AUG)
    als1_f = jnp.pad(als1.T, ((0, 0), (0, NPAD - N))).reshape(-1)
    ald1_f = jnp.pad(ald1.T, ((0, 0), (0, NPAD - N))).reshape(-1)

    acc1 = _sc_l1(haug1_f, s1, d1, als1_f, ald1_f,
                  mxs1.reshape(-1), mxd1.reshape(-1), zrs)
    acc1 = acc1.reshape(HEADS, NPAD, AUG)

    haug2, haug3, al23, mx23 = _tc_mid(
        acc1, b1.reshape(1, -1), W2, a_src2, a_dst2, W3, a_src3, a_dst3)
    haug23_f = jnp.concatenate([haug2, haug3], axis=0)
    al23_f = jnp.pad(al23.T, ((0, 0), (0, NPAD - N))).reshape(-1)
    s23 = jnp.concatenate([s1, s2])
    d23 = jnp.concatenate([d1, d2])

    acc23 = _sc_l23(haug23_f, s23, d23, al23_f, mx23.reshape(-1), zrs)

    logits, logits2, preds = _tc_post(acc23[:NPAD], acc23[NPAD:],
                                      b2.reshape(1, -1), b3.reshape(1, -1))
    return (logits, logits2, preds.reshape(-1))


# speedup vs baseline: 29.1106x; 1.5079x over previous
import functools

import jax
import jax.numpy as jnp
from jax import lax
from jax.experimental import pallas as pl
from jax.experimental.pallas import tpu as pltpu
from jax.experimental.pallas import tpu_sc as plsc

N = 10000
E = 320000
EP = E + N
F_IN = 128
HID = 64
HEADS = 8
OUT = 64
AUG = 128

NTILES = 16
B = 64
PER_TILE = 20736
EPAD = PER_TILE * NTILES
NPAD = 10240
RPT = NPAD // NTILES
NB = 10
BN = N // NB

_f32 = jnp.float32
_i32 = jnp.int32


def _tc_pre1_body(x_ref, w_ref, asr_ref, adr_ref,
                  haug_ref, als_ref, ald_ref, mxs_ref, mxd_ref):
    i = pl.program_id(0)

    @pl.when(i == 0)
    def _():
        mxs_ref[...] = jnp.full((HEADS, 128), -1e30, _f32)
        mxd_ref[...] = jnp.full((HEADS, 128), -1e30, _f32)

    h = jnp.dot(x_ref[...], w_ref[...], preferred_element_type=_f32)
    ones = jnp.ones((BN, 1), _f32)
    zpad = jnp.zeros((BN, AUG - HID - 1), _f32)
    a_s_all, a_d_all = [], []
    for hd in range(HEADS):
        hh = h[:, hd * HID:(hd + 1) * HID]
        a_s = jnp.sum(hh * asr_ref[hd, :][None, :], axis=1)
        a_d = jnp.sum(hh * adr_ref[hd, :][None, :], axis=1)
        a_s_all.append(a_s)
        a_d_all.append(a_d)
        mxs_ref[hd, :] = jnp.maximum(mxs_ref[hd, :], jnp.max(a_s))
        mxd_ref[hd, :] = jnp.maximum(mxd_ref[hd, :], jnp.max(a_d))
        haug_ref[hd, :, :] = jnp.concatenate([hh, ones, zpad], axis=1)
    als_ref[...] = jnp.stack(a_s_all, axis=1)
    ald_ref[...] = jnp.stack(a_d_all, axis=1)


def _tc_pre1(x, W1, a_src1, a_dst1):
    return pl.pallas_call(
        _tc_pre1_body,
        grid=(NB,),
        in_specs=[
            pl.BlockSpec((BN, F_IN), lambda i: (i, 0)),
            pl.BlockSpec((F_IN, HEADS * HID), lambda i: (0, 0)),
            pl.BlockSpec((HEADS, HID), lambda i: (0, 0)),
            pl.BlockSpec((HEADS, HID), lambda i: (0, 0)),
        ],
        out_specs=[
            pl.BlockSpec((HEADS, BN, AUG), lambda i: (0, i, 0)),
            pl.BlockSpec((BN, HEADS), lambda i: (i, 0)),
            pl.BlockSpec((BN, HEADS), lambda i: (i, 0)),
            pl.BlockSpec((HEADS, 128), lambda i: (0, 0)),
            pl.BlockSpec((HEADS, 128), lambda i: (0, 0)),
        ],
        out_shape=[
            jax.ShapeDtypeStruct((HEADS, N, AUG), _f32),
            jax.ShapeDtypeStruct((N, HEADS), _f32),
            jax.ShapeDtypeStruct((N, HEADS), _f32),
            jax.ShapeDtypeStruct((HEADS, 128), _f32),
            jax.ShapeDtypeStruct((HEADS, 128), _f32),
        ],
    )(x, W1, a_src1, a_dst1)


NCH = PER_TILE // B
NSC = (HID + 16) // 16


def _sc_chunk_loop(haug, sd, acc, tab_s, tab_d, sdbuf, dvec, gidx,
                   rows, eev, isem, gsem, ssem, bnd_v, cbase, goff):
    def idx_issue(c, b):
        off = (cbase + c) * (2 * B)
        pltpu.async_copy(sd.at[pl.ds(off, 2 * B)], sdbuf[b], isem[b])

    def prep(c, b):
        pltpu.make_async_copy(
            sd.at[pl.ds((cbase + c) * (2 * B), 2 * B)],
            sdbuf[b], isem[b]).wait()

        def sub16(k, _):
            o = k * 16
            sid = sdbuf[b][pl.ds(o, 16)]
            did = sdbuf[b][pl.ds(B + o, 16)]
            gidx[b][pl.ds(o, 16)] = sid + goff
            dvec[b][pl.ds(o, 16)] = did
            e = (plsc.load_gather(tab_s, [sid])
                 + plsc.load_gather(tab_d, [did]))
            e = jnp.where(e > 0, e, 0.2 * e)
            eev[b][pl.ds(o, 16)] = jnp.exp(e - bnd_v[...])
            return 0

        lax.fori_loop(0, B // 16, sub16, 0, unroll=2)
        pltpu.async_copy(haug.at[gidx[b]], rows[b], gsem[b])

    def drain(b):
        pltpu.make_async_copy(rows[b], acc.at[dvec[b]], ssem[b]).wait()

    def finish(b):
        pltpu.make_async_copy(haug.at[gidx[b]], rows[b], gsem[b]).wait()

        def scale(j, _):
            ej = plsc.load_gather(
                eev[b], [jnp.broadcast_to(j, (16,)).astype(_i32)])
            for q in range(NSC):
                rows[b][j, pl.ds(q * 16, 16)] = \
                    rows[b][j, pl.ds(q * 16, 16)] * ej
            return 0

        lax.fori_loop(0, B, scale, 0, unroll=4)
        pltpu.async_copy(rows[b], acc.at[dvec[b]], ssem[b], add=True)

    idx_issue(0, 0)
    idx_issue(1, 1)
    prep(0, 0)

    def outer(g, _):
        for b in range(2):
            c = 2 * g + b

            @pl.when(c + 2 < NCH)
            def _():
                idx_issue(c + 2, b)

            @pl.when(c + 1 < NCH)
            def _():
                @pl.when(c >= 1)
                def _():
                    drain(1 - b)
                prep(c + 1, 1 - b)

            finish(b)
        return 0

    lax.fori_loop(0, NCH // 2, outer, 0)
    drain(0)
    drain(1)


def _sc_l1_body(haug, sd, alps, alpd, mxs, mxd, zrs, out,
                tab_s, tab_d, sdbuf0, sdbuf1, dvec0, dvec1, gidx0, gidx1,
                rows0, rows1, eev0, eev1,
                m1v, m2v, bnd_v, acc, isem0, isem1,
                gsem0, gsem1, ssem0, ssem1):
    core = lax.axis_index("c")
    sub = lax.axis_index("s")
    cbase = sub * NCH
    rbase = sub * RPT
    for hl in range(HEADS // 2):
        head = core * (HEADS // 2) + hl
        pltpu.sync_copy(zrs, acc.at[pl.ds(rbase, RPT)])
        pltpu.sync_copy(alps.at[pl.ds(head * NPAD, NPAD)], tab_s)
        pltpu.sync_copy(alpd.at[pl.ds(head * NPAD, NPAD)], tab_d)
        pltpu.sync_copy(mxs.at[pl.ds(head * 128, 16)], m1v)
        pltpu.sync_copy(mxd.at[pl.ds(head * 128, 16)], m2v)
        b = m1v[...] + m2v[...]
        bnd_v[...] = jnp.where(b > 0, b, 0.2 * b)
        plsc.subcore_barrier()
        _sc_chunk_loop(haug, sd, acc, tab_s, tab_d,
                       (sdbuf0, sdbuf1), (dvec0, dvec1), (gidx0, gidx1),
                       (rows0, rows1), (eev0, eev1),
                       (isem0, isem1), (gsem0, gsem1), (ssem0, ssem1),
                       bnd_v, cbase, head * N)
        plsc.subcore_barrier()
        pltpu.sync_copy(acc.at[pl.ds(rbase, RPT)],
                        out.at[pl.ds(head * NPAD + rbase, RPT)])
        plsc.subcore_barrier()


def _sc_l1(haug_f, sd, alps_f, alpd_f, mxs_f, mxd_f, zrs):
    mesh = plsc.VectorSubcoreMesh(core_axis_name="c", subcore_axis_name="s")
    return pl.kernel(
        _sc_l1_body,
        out_type=jax.ShapeDtypeStruct((HEADS * NPAD, AUG), _f32),
        mesh=mesh,
        compiler_params=pltpu.CompilerParams(needs_layout_passes=False),
        scratch_types=_sc_scratch(),
    )(haug_f, sd, alps_f, alpd_f, mxs_f, mxd_f, zrs)


def _sc_scratch():
    return [
        pltpu.VMEM((NPAD,), _f32),
        pltpu.VMEM((NPAD,), _f32),
        pltpu.VMEM((2 * B,), _i32),
        pltpu.VMEM((2 * B,), _i32),
        pltpu.VMEM((B,), _i32),
        pltpu.VMEM((B,), _i32),
        pltpu.VMEM((B,), _i32),
        pltpu.VMEM((B,), _i32),
        pltpu.VMEM((B, AUG), _f32),
        pltpu.VMEM((B, AUG), _f32),
        pltpu.VMEM((B,), _f32),
        pltpu.VMEM((B,), _f32),
        pltpu.VMEM((16,), _f32),
        pltpu.VMEM((16,), _f32),
        pltpu.VMEM((16,), _f32),
        pltpu.VMEM_SHARED((NPAD, AUG), _f32),
        pltpu.SemaphoreType.DMA,
        pltpu.SemaphoreType.DMA,
        pltpu.SemaphoreType.DMA,
        pltpu.SemaphoreType.DMA,
        pltpu.SemaphoreType.DMA,
        pltpu.SemaphoreType.DMA,
    ]


def _sc_l23_body(haug, sd, alps, mxs, zrs, out,
                 tab_s, tab_d, sdbuf0, sdbuf1, dvec0, dvec1, gidx0, gidx1,
                 rows0, rows1, eev0, eev1,
                 m1v, m2v, bnd_v, acc, isem0, isem1,
                 gsem0, gsem1, ssem0, ssem1):
    core = lax.axis_index("c")
    sub = lax.axis_index("s")
    cbase = core * (EPAD // B) + sub * NCH
    rbase = sub * RPT
    row_s = 2 * core
    row_d = 2 * core + 1
    pltpu.sync_copy(zrs, acc.at[pl.ds(rbase, RPT)])
    pltpu.sync_copy(alps.at[pl.ds(row_s * NPAD, NPAD)], tab_s)
    pltpu.sync_copy(alps.at[pl.ds(row_d * NPAD, NPAD)], tab_d)
    pltpu.sync_copy(mxs.at[pl.ds(row_s * 128, 16)], m1v)
    pltpu.sync_copy(mxs.at[pl.ds(row_d * 128, 16)], m2v)
    b = m1v[...] + m2v[...]
    bnd_v[...] = jnp.where(b > 0, b, 0.2 * b)
    plsc.subcore_barrier()
    _sc_chunk_loop(haug, sd, acc, tab_s, tab_d,
                   (sdbuf0, sdbuf1), (dvec0, dvec1), (gidx0, gidx1),
                   (rows0, rows1), (eev0, eev1),
                   (isem0, isem1), (gsem0, gsem1), (ssem0, ssem1),
                   bnd_v, cbase, core * N)
    plsc.subcore_barrier()
    pltpu.sync_copy(acc.at[pl.ds(rbase, RPT)],
                    out.at[pl.ds(core * NPAD + rbase, RPT)])


def _sc_l23(haug_f, sd2, alps_f, mxs_f, zrs):
    mesh = plsc.VectorSubcoreMesh(core_axis_name="c", subcore_axis_name="s")
    return pl.kernel(
        _sc_l23_body,
        out_type=jax.ShapeDtypeStruct((2 * NPAD, AUG), _f32),
        mesh=mesh,
        compiler_params=pltpu.CompilerParams(needs_layout_passes=False),
        scratch_types=_sc_scratch(),
    )(haug_f, sd2, alps_f, mxs_f, zrs)


def _tc_mid_body(acc_ref, b1_ref, w2_ref, as2_ref, ad2_ref,
                 w3_ref, as3_ref, ad3_ref,
                 haug2_ref, haug3_ref, al_ref, mx_ref):
    i = pl.program_id(0)

    @pl.when(i == 0)
    def _():
        mx_ref[...] = jnp.full((HEADS, 128), -1e30, _f32)

    cols = []
    for hd in range(HEADS):
        num = acc_ref[hd, :, 0:HID]
        den = acc_ref[hd, :, HID:HID + 1]
        cols.append(num / (den + 1e-16))
    emb = jnp.concatenate(cols, axis=1) + b1_ref[0, :][None, :]
    emb = jnp.where(emb > 0, emb, jnp.exp(jnp.minimum(emb, 0.0)) - 1.0)

    ones = jnp.ones((BN, 1), _f32)
    zpad = jnp.zeros((BN, AUG - OUT - 1), _f32)

    h2 = jnp.dot(emb, w2_ref[...], preferred_element_type=_f32)
    a_s2 = jnp.sum(h2 * as2_ref[0, :][None, :], axis=1)
    a_d2 = jnp.sum(h2 * ad2_ref[0, :][None, :], axis=1)
    haug2_ref[...] = jnp.concatenate([h2, ones, zpad], axis=1)

    h3 = jnp.dot(emb, w3_ref[...], preferred_element_type=_f32)
    a_s3 = jnp.sum(h3 * as3_ref[0, :][None, :], axis=1)
    a_d3 = jnp.sum(h3 * ad3_ref[0, :][None, :], axis=1)
    haug3_ref[...] = jnp.concatenate([h3, ones, zpad], axis=1)

    zcol = jnp.zeros((BN,), _f32)
    al_ref[...] = jnp.stack(
        [a_s2, a_d2, a_s3, a_d3] + [zcol] * (HEADS - 4), axis=1)
    mx_ref[0, :] = jnp.maximum(mx_ref[0, :], jnp.max(a_s2))
    mx_ref[1, :] = jnp.maximum(mx_ref[1, :], jnp.max(a_d2))
    mx_ref[2, :] = jnp.maximum(mx_ref[2, :], jnp.max(a_s3))
    mx_ref[3, :] = jnp.maximum(mx_ref[3, :], jnp.max(a_d3))


def _tc_mid(acc1, b1, W2, a_src2, a_dst2, W3, a_src3, a_dst3):
    return pl.pallas_call(
        _tc_mid_body,
        grid=(NB,),
        in_specs=[
            pl.BlockSpec((HEADS, BN, AUG), lambda i: (0, i, 0)),
            pl.BlockSpec((1, HEADS * HID), lambda i: (0, 0)),
            pl.BlockSpec((HEADS * HID, OUT), lambda i: (0, 0)),
            pl.BlockSpec((1, OUT), lambda i: (0, 0)),
            pl.BlockSpec((1, OUT), lambda i: (0, 0)),
            pl.BlockSpec((HEADS * HID, OUT), lambda i: (0, 0)),
            pl.BlockSpec((1, OUT), lambda i: (0, 0)),
            pl.BlockSpec((1, OUT), lambda i: (0, 0)),
        ],
        out_specs=[
            pl.BlockSpec((BN, AUG), lambda i: (i, 0)),
            pl.BlockSpec((BN, AUG), lambda i: (i, 0)),
            pl.BlockSpec((BN, HEADS), lambda i: (i, 0)),
            pl.BlockSpec((HEADS, 128), lambda i: (0, 0)),
        ],
        out_shape=[
            jax.ShapeDtypeStruct((N, AUG), _f32),
            jax.ShapeDtypeStruct((N, AUG), _f32),
            jax.ShapeDtypeStruct((N, HEADS), _f32),
            jax.ShapeDtypeStruct((HEADS, 128), _f32),
        ],
    )(acc1, b1, W2, a_src2, a_dst2, W3, a_src3, a_dst3)


def _tc_post_body(acc2_ref, acc3_ref, b2_ref, b3_ref,
                  lg1_ref, lg2_ref, prd_ref):
    x1 = acc2_ref[:, 0:OUT] / (acc2_ref[:, HID:HID + 1] + 1e-16) \
        + b2_ref[0, :][None, :]
    x2 = acc3_ref[:, 0:OUT] / (acc3_ref[:, HID:HID + 1] + 1e-16) \
        + b3_ref[0, :][None, :]
    m1 = jnp.max(x1, axis=1, keepdims=True)
    p1 = jnp.exp(x1 - m1)
    lg1_ref[...] = p1 / jnp.sum(p1, axis=1, keepdims=True)
    m2 = jnp.max(x2, axis=1, keepdims=True)
    p2 = jnp.exp(x2 - m2)
    lg2_ref[...] = p2 / jnp.sum(p2, axis=1, keepdims=True)
    ii = lax.broadcasted_iota(_i32, (BN, OUT), 1)
    cand = jnp.where(x1 == m1, ii, OUT)
    prd_ref[0, 0, :] = jnp.min(cand, axis=1)


def _tc_post(acc2, acc3, b2, b3):
    return pl.pallas_call(
        _tc_post_body,
        grid=(NB,),
        in_specs=[
            pl.BlockSpec((BN, AUG), lambda i: (i, 0)),
            pl.BlockSpec((BN, AUG), lambda i: (i, 0)),
            pl.BlockSpec((1, OUT), lambda i: (0, 0)),
            pl.BlockSpec((1, OUT), lambda i: (0, 0)),
        ],
        out_specs=[
            pl.BlockSpec((BN, OUT), lambda i: (i, 0)),
            pl.BlockSpec((BN, OUT), lambda i: (i, 0)),
            pl.BlockSpec((1, 1, BN), lambda i: (i, 0, 0)),
        ],
        out_shape=[
            jax.ShapeDtypeStruct((N, OUT), _f32),
            jax.ShapeDtypeStruct((N, OUT), _f32),
            jax.ShapeDtypeStruct((NB, 1, BN), _i32),
        ],
    )(acc2, acc3, b2, b3)


def _pad_edges(src, dst):
    loop = jnp.arange(N, dtype=_i32)
    npad = EPAD - EP
    s = jnp.concatenate([src.astype(_i32), loop,
                         jnp.zeros((npad,), _i32)])
    d = jnp.concatenate([dst.astype(_i32), loop,
                         jnp.full((npad,), N, _i32)])
    return jnp.stack([s.reshape(-1, B), d.reshape(-1, B)],
                     axis=1).reshape(-1)


def kernel(x, edge_index, edge_index_2, W1, a_src1, a_dst1, b1,
           W2, a_src2, a_dst2, b2, W3, a_src3, a_dst3, b3):
    sd1 = _pad_edges(edge_index[0], edge_index[1])
    sd2 = _pad_edges(edge_index_2[0], edge_index_2[1])
    zrs = jnp.zeros((RPT, AUG), _f32)

    haug1, als1, ald1, mxs1, mxd1 = _tc_pre1(x, W1, a_src1, a_dst1)
    haug1_f = haug1.reshape(HEADS * N, AUG)
    als1_f = jnp.pad(als1.T, ((0, 0), (0, NPAD - N))).reshape(-1)
    ald1_f = jnp.pad(ald1.T, ((0, 0), (0, NPAD - N))).reshape(-1)

    acc1 = _sc_l1(haug1_f, sd1, als1_f, ald1_f,
                  mxs1.reshape(-1), mxd1.reshape(-1), zrs)
    acc1 = acc1.reshape(HEADS, NPAD, AUG)

    haug2, haug3, al23, mx23 = _tc_mid(
        acc1, b1.reshape(1, -1), W2, a_src2, a_dst2, W3, a_src3, a_dst3)
    haug23_f = jnp.concatenate([haug2, haug3], axis=0)
    al23_f = jnp.pad(al23.T, ((0, 0), (0, NPAD - N))).reshape(-1)
    sd23 = jnp.concatenate([sd1, sd2])

    acc23 = _sc_l23(haug23_f, sd23, al23_f, mx23.reshape(-1), zrs)

    logits, logits2, preds = _tc_post(acc23[:NPAD], acc23[NPAD:],
                                      b2.reshape(1, -1), b3.reshape(1, -1))
    return (logits, logits2, preds.reshape(-1))


# speedup vs baseline: 29.6356x; 1.0180x over previous
import functools

import jax
import jax.numpy as jnp
from jax import lax
from jax.experimental import pallas as pl
from jax.experimental.pallas import tpu as pltpu
from jax.experimental.pallas import tpu_sc as plsc

N = 10000
E = 320000
EP = E + N
F_IN = 128
HID = 64
HEADS = 8
OUT = 64
AUG = 128

NTILES = 16
B = 64
PER_TILE = 20736
EPAD = PER_TILE * NTILES
NPAD = 10240
RPT = NPAD // NTILES
NB = 10
BN = N // NB

_f32 = jnp.float32
_i32 = jnp.int32


def _tc_pre1_body(x_ref, w_ref, asr_ref, adr_ref,
                  haug_ref, als_ref, ald_ref, mxs_ref, mxd_ref):
    i = pl.program_id(0)

    @pl.when(i == 0)
    def _():
        mxs_ref[...] = jnp.full((HEADS, 128), -1e30, _f32)
        mxd_ref[...] = jnp.full((HEADS, 128), -1e30, _f32)

    h = jnp.dot(x_ref[...], w_ref[...], preferred_element_type=_f32)
    ones = jnp.ones((BN, 1), _f32)
    zpad = jnp.zeros((BN, AUG - HID - 1), _f32)
    a_s_all, a_d_all = [], []
    for hd in range(HEADS):
        hh = h[:, hd * HID:(hd + 1) * HID]
        a_s = jnp.sum(hh * asr_ref[hd, :][None, :], axis=1)
        a_d = jnp.sum(hh * adr_ref[hd, :][None, :], axis=1)
        a_s_all.append(a_s)
        a_d_all.append(a_d)
        mxs_ref[hd, :] = jnp.maximum(mxs_ref[hd, :], jnp.max(a_s))
        mxd_ref[hd, :] = jnp.maximum(mxd_ref[hd, :], jnp.max(a_d))
        haug_ref[hd, :, :] = jnp.concatenate([hh, ones, zpad], axis=1)
    als_ref[...] = jnp.stack(a_s_all, axis=1)
    ald_ref[...] = jnp.stack(a_d_all, axis=1)


def _tc_pre1(x, W1, a_src1, a_dst1):
    return pl.pallas_call(
        _tc_pre1_body,
        grid=(NB,),
        in_specs=[
            pl.BlockSpec((BN, F_IN), lambda i: (i, 0)),
            pl.BlockSpec((F_IN, HEADS * HID), lambda i: (0, 0)),
            pl.BlockSpec((HEADS, HID), lambda i: (0, 0)),
            pl.BlockSpec((HEADS, HID), lambda i: (0, 0)),
        ],
        out_specs=[
            pl.BlockSpec((HEADS, BN, AUG), lambda i: (0, i, 0)),
            pl.BlockSpec((BN, HEADS), lambda i: (i, 0)),
            pl.BlockSpec((BN, HEADS), lambda i: (i, 0)),
            pl.BlockSpec((HEADS, 128), lambda i: (0, 0)),
            pl.BlockSpec((HEADS, 128), lambda i: (0, 0)),
        ],
        out_shape=[
            jax.ShapeDtypeStruct((HEADS, N, AUG), _f32),
            jax.ShapeDtypeStruct((N, HEADS), _f32),
            jax.ShapeDtypeStruct((N, HEADS), _f32),
            jax.ShapeDtypeStruct((HEADS, 128), _f32),
            jax.ShapeDtypeStruct((HEADS, 128), _f32),
        ],
    )(x, W1, a_src1, a_dst1)


NCH = PER_TILE // B
NSC = (HID + 16) // 16


def _sc_chunk_loop(haug, sd, acc, tab_s, tab_d, sdbuf, dvec, gidx,
                   rows, eev, isem, gsem, ssem, bnd_v, cbase, goff):
    def idx_issue(c, s):
        off = (cbase + c) * (2 * B)
        pltpu.async_copy(sd.at[pl.ds(off, 2 * B)], sdbuf[s], isem[s])

    def prep(c, s):
        pltpu.make_async_copy(
            sd.at[pl.ds((cbase + c) * (2 * B), 2 * B)],
            sdbuf[s], isem[s]).wait()

        def sub16(k, _):
            o = k * 16
            sid = sdbuf[s][pl.ds(o, 16)]
            did = sdbuf[s][pl.ds(B + o, 16)]
            gidx[s][pl.ds(o, 16)] = sid + goff
            dvec[s][pl.ds(o, 16)] = did
            e = (plsc.load_gather(tab_s, [sid])
                 + plsc.load_gather(tab_d, [did]))
            e = jnp.where(e > 0, e, 0.2 * e)
            eev[s][pl.ds(o, 16)] = jnp.exp(e - bnd_v[...])
            return 0

        lax.fori_loop(0, B // 16, sub16, 0, unroll=2)
        pltpu.async_copy(haug.at[gidx[s]], rows[s], gsem[s])

    def drain(s):
        pltpu.make_async_copy(rows[s], acc.at[dvec[s]], ssem[s]).wait()

    def finish(s):
        pltpu.make_async_copy(haug.at[gidx[s]], rows[s], gsem[s]).wait()

        def scale(j, _):
            ej = plsc.load_gather(
                eev[s], [jnp.broadcast_to(j, (16,)).astype(_i32)])
            for q in range(NSC):
                rows[s][j, pl.ds(q * 16, 16)] = \
                    rows[s][j, pl.ds(q * 16, 16)] * ej
            return 0

        lax.fori_loop(0, B, scale, 0, unroll=4)
        pltpu.async_copy(rows[s], acc.at[dvec[s]], ssem[s], add=True)

    idx_issue(0, 0)
    idx_issue(1, 1)
    idx_issue(2, 2)
    prep(0, 0)
    prep(1, 1)

    def outer(g, _):
        for k in range(3):
            c = 3 * g + k
            s0 = k
            s2 = (k + 2) % 3

            @pl.when(c + 3 < NCH)
            def _():
                idx_issue(c + 3, s0)

            @pl.when(c + 2 < NCH)
            def _():
                @pl.when(c >= 1)
                def _():
                    drain(s2)
                prep(c + 2, s2)

            finish(s0)
        return 0

    lax.fori_loop(0, NCH // 3, outer, 0)
    drain(0)
    drain(1)
    drain(2)


def _sc_l1_body(haug, sd, alps, alpd, mxs, mxd, zrs, out,
                tab_s, tab_d, sdbuf0, sdbuf1, sdbuf2,
                dvec0, dvec1, dvec2, gidx0, gidx1, gidx2,
                rows0, rows1, rows2, eev0, eev1, eev2,
                m1v, m2v, bnd_v, acc, isem0, isem1, isem2,
                gsem0, gsem1, gsem2, ssem0, ssem1, ssem2):
    core = lax.axis_index("c")
    sub = lax.axis_index("s")
    cbase = sub * NCH
    rbase = sub * RPT
    for hl in range(HEADS // 2):
        head = core * (HEADS // 2) + hl
        pltpu.sync_copy(zrs, acc.at[pl.ds(rbase, RPT)])
        pltpu.sync_copy(alps.at[pl.ds(head * NPAD, NPAD)], tab_s)
        pltpu.sync_copy(alpd.at[pl.ds(head * NPAD, NPAD)], tab_d)
        pltpu.sync_copy(mxs.at[pl.ds(head * 128, 16)], m1v)
        pltpu.sync_copy(mxd.at[pl.ds(head * 128, 16)], m2v)
        b = m1v[...] + m2v[...]
        bnd_v[...] = jnp.where(b > 0, b, 0.2 * b)
        plsc.subcore_barrier()
        _sc_chunk_loop(haug, sd, acc, tab_s, tab_d,
                       (sdbuf0, sdbuf1, sdbuf2), (dvec0, dvec1, dvec2),
                       (gidx0, gidx1, gidx2), (rows0, rows1, rows2),
                       (eev0, eev1, eev2), (isem0, isem1, isem2),
                       (gsem0, gsem1, gsem2), (ssem0, ssem1, ssem2),
                       bnd_v, cbase, head * N)
        plsc.subcore_barrier()
        pltpu.sync_copy(acc.at[pl.ds(rbase, RPT)],
                        out.at[pl.ds(head * NPAD + rbase, RPT)])
        plsc.subcore_barrier()


def _sc_l1(haug_f, sd, alps_f, alpd_f, mxs_f, mxd_f, zrs):
    mesh = plsc.VectorSubcoreMesh(core_axis_name="c", subcore_axis_name="s")
    return pl.kernel(
        _sc_l1_body,
        out_type=jax.ShapeDtypeStruct((HEADS * NPAD, AUG), _f32),
        mesh=mesh,
        compiler_params=pltpu.CompilerParams(needs_layout_passes=False),
        scratch_types=_sc_scratch(),
    )(haug_f, sd, alps_f, alpd_f, mxs_f, mxd_f, zrs)


def _sc_scratch():
    ring = lambda t: [t, t, t]
    return (
        [pltpu.VMEM((NPAD,), _f32),
         pltpu.VMEM((NPAD,), _f32)]
        + ring(pltpu.VMEM((2 * B,), _i32))
        + ring(pltpu.VMEM((B,), _i32))
        + ring(pltpu.VMEM((B,), _i32))
        + ring(pltpu.VMEM((B, AUG), _f32))
        + ring(pltpu.VMEM((B,), _f32))
        + [pltpu.VMEM((16,), _f32),
           pltpu.VMEM((16,), _f32),
           pltpu.VMEM((16,), _f32),
           pltpu.VMEM_SHARED((NPAD, AUG), _f32)]
        + [pltpu.SemaphoreType.DMA] * 9
    )


def _sc_l23_body(haug, sd, alps, mxs, zrs, out,
                 tab_s, tab_d, sdbuf0, sdbuf1, sdbuf2,
                 dvec0, dvec1, dvec2, gidx0, gidx1, gidx2,
                 rows0, rows1, rows2, eev0, eev1, eev2,
                 m1v, m2v, bnd_v, acc, isem0, isem1, isem2,
                 gsem0, gsem1, gsem2, ssem0, ssem1, ssem2):
    core = lax.axis_index("c")
    sub = lax.axis_index("s")
    cbase = core * (EPAD // B) + sub * NCH
    rbase = sub * RPT
    row_s = 2 * core
    row_d = 2 * core + 1
    pltpu.sync_copy(zrs, acc.at[pl.ds(rbase, RPT)])
    pltpu.sync_copy(alps.at[pl.ds(row_s * NPAD, NPAD)], tab_s)
    pltpu.sync_copy(alps.at[pl.ds(row_d * NPAD, NPAD)], tab_d)
    pltpu.sync_copy(mxs.at[pl.ds(row_s * 128, 16)], m1v)
    pltpu.sync_copy(mxs.at[pl.ds(row_d * 128, 16)], m2v)
    b = m1v[...] + m2v[...]
    bnd_v[...] = jnp.where(b > 0, b, 0.2 * b)
    plsc.subcore_barrier()
    _sc_chunk_loop(haug, sd, acc, tab_s, tab_d,
                   (sdbuf0, sdbuf1, sdbuf2), (dvec0, dvec1, dvec2),
                   (gidx0, gidx1, gidx2), (rows0, rows1, rows2),
                   (eev0, eev1, eev2), (isem0, isem1, isem2),
                   (gsem0, gsem1, gsem2), (ssem0, ssem1, ssem2),
                   bnd_v, cbase, core * N)
    plsc.subcore_barrier()
    pltpu.sync_copy(acc.at[pl.ds(rbase, RPT)],
                    out.at[pl.ds(core * NPAD + rbase, RPT)])


def _sc_l23(haug_f, sd2, alps_f, mxs_f, zrs):
    mesh = plsc.VectorSubcoreMesh(core_axis_name="c", subcore_axis_name="s")
    return pl.kernel(
        _sc_l23_body,
        out_type=jax.ShapeDtypeStruct((2 * NPAD, AUG), _f32),
        mesh=mesh,
        compiler_params=pltpu.CompilerParams(needs_layout_passes=False),
        scratch_types=_sc_scratch(),
    )(haug_f, sd2, alps_f, mxs_f, zrs)


def _tc_mid_body(acc_ref, b1_ref, w2_ref, as2_ref, ad2_ref,
                 w3_ref, as3_ref, ad3_ref,
                 haug2_ref, haug3_ref, al_ref, mx_ref):
    i = pl.program_id(0)

    @pl.when(i == 0)
    def _():
        mx_ref[...] = jnp.full((HEADS, 128), -1e30, _f32)

    cols = []
    for hd in range(HEADS):
        num = acc_ref[hd, :, 0:HID]
        den = acc_ref[hd, :, HID:HID + 1]
        cols.append(num / (den + 1e-16))
    emb = jnp.concatenate(cols, axis=1) + b1_ref[0, :][None, :]
    emb = jnp.where(emb > 0, emb, jnp.exp(jnp.minimum(emb, 0.0)) - 1.0)

    ones = jnp.ones((BN, 1), _f32)
    zpad = jnp.zeros((BN, AUG - OUT - 1), _f32)

    h2 = jnp.dot(emb, w2_ref[...], preferred_element_type=_f32)
    a_s2 = jnp.sum(h2 * as2_ref[0, :][None, :], axis=1)
    a_d2 = jnp.sum(h2 * ad2_ref[0, :][None, :], axis=1)
    haug2_ref[...] = jnp.concatenate([h2, ones, zpad], axis=1)

    h3 = jnp.dot(emb, w3_ref[...], preferred_element_type=_f32)
    a_s3 = jnp.sum(h3 * as3_ref[0, :][None, :], axis=1)
    a_d3 = jnp.sum(h3 * ad3_ref[0, :][None, :], axis=1)
    haug3_ref[...] = jnp.concatenate([h3, ones, zpad], axis=1)

    zcol = jnp.zeros((BN,), _f32)
    al_ref[...] = jnp.stack(
        [a_s2, a_d2, a_s3, a_d3] + [zcol] * (HEADS - 4), axis=1)
    mx_ref[0, :] = jnp.maximum(mx_ref[0, :], jnp.max(a_s2))
    mx_ref[1, :] = jnp.maximum(mx_ref[1, :], jnp.max(a_d2))
    mx_ref[2, :] = jnp.maximum(mx_ref[2, :], jnp.max(a_s3))
    mx_ref[3, :] = jnp.maximum(mx_ref[3, :], jnp.max(a_d3))


def _tc_mid(acc1, b1, W2, a_src2, a_dst2, W3, a_src3, a_dst3):
    return pl.pallas_call(
        _tc_mid_body,
        grid=(NB,),
        in_specs=[
            pl.BlockSpec((HEADS, BN, AUG), lambda i: (0, i, 0)),
            pl.BlockSpec((1, HEADS * HID), lambda i: (0, 0)),
            pl.BlockSpec((HEADS * HID, OUT), lambda i: (0, 0)),
            pl.BlockSpec((1, OUT), lambda i: (0, 0)),
            pl.BlockSpec((1, OUT), lambda i: (0, 0)),
            pl.BlockSpec((HEADS * HID, OUT), lambda i: (0, 0)),
            pl.BlockSpec((1, OUT), lambda i: (0, 0)),
            pl.BlockSpec((1, OUT), lambda i: (0, 0)),
        ],
        out_specs=[
            pl.BlockSpec((BN, AUG), lambda i: (i, 0)),
            pl.BlockSpec((BN, AUG), lambda i: (i, 0)),
            pl.BlockSpec((BN, HEADS), lambda i: (i, 0)),
            pl.BlockSpec((HEADS, 128), lambda i: (0, 0)),
        ],
        out_shape=[
            jax.ShapeDtypeStruct((N, AUG), _f32),
            jax.ShapeDtypeStruct((N, AUG), _f32),
            jax.ShapeDtypeStruct((N, HEADS), _f32),
            jax.ShapeDtypeStruct((HEADS, 128), _f32),
        ],
    )(acc1, b1, W2, a_src2, a_dst2, W3, a_src3, a_dst3)


def _tc_post_body(acc2_ref, acc3_ref, b2_ref, b3_ref,
                  lg1_ref, lg2_ref, prd_ref):
    x1 = acc2_ref[:, 0:OUT] / (acc2_ref[:, HID:HID + 1] + 1e-16) \
        + b2_ref[0, :][None, :]
    x2 = acc3_ref[:, 0:OUT] / (acc3_ref[:, HID:HID + 1] + 1e-16) \
        + b3_ref[0, :][None, :]
    m1 = jnp.max(x1, axis=1, keepdims=True)
    p1 = jnp.exp(x1 - m1)
    lg1_ref[...] = p1 / jnp.sum(p1, axis=1, keepdims=True)
    m2 = jnp.max(x2, axis=1, keepdims=True)
    p2 = jnp.exp(x2 - m2)
    lg2_ref[...] = p2 / jnp.sum(p2, axis=1, keepdims=True)
    ii = lax.broadcasted_iota(_i32, (BN, OUT), 1)
    cand = jnp.where(x1 == m1, ii, OUT)
    prd_ref[0, 0, :] = jnp.min(cand, axis=1)


def _tc_post(acc2, acc3, b2, b3):
    return pl.pallas_call(
        _tc_post_body,
        grid=(NB,),
        in_specs=[
            pl.BlockSpec((BN, AUG), lambda i: (i, 0)),
            pl.BlockSpec((BN, AUG), lambda i: (i, 0)),
            pl.BlockSpec((1, OUT), lambda i: (0, 0)),
            pl.BlockSpec((1, OUT), lambda i: (0, 0)),
        ],
        out_specs=[
            pl.BlockSpec((BN, OUT), lambda i: (i, 0)),
            pl.BlockSpec((BN, OUT), lambda i: (i, 0)),
            pl.BlockSpec((1, 1, BN), lambda i: (i, 0, 0)),
        ],
        out_shape=[
            jax.ShapeDtypeStruct((N, OUT), _f32),
            jax.ShapeDtypeStruct((N, OUT), _f32),
            jax.ShapeDtypeStruct((NB, 1, BN), _i32),
        ],
    )(acc2, acc3, b2, b3)


def _pad_edges(src, dst):
    loop = jnp.arange(N, dtype=_i32)
    npad = EPAD - EP
    s = jnp.concatenate([src.astype(_i32), loop,
                         jnp.zeros((npad,), _i32)])
    d = jnp.concatenate([dst.astype(_i32), loop,
                         jnp.full((npad,), N, _i32)])
    return jnp.stack([s.reshape(-1, B), d.reshape(-1, B)],
                     axis=1).reshape(-1)


def kernel(x, edge_index, edge_index_2, W1, a_src1, a_dst1, b1,
           W2, a_src2, a_dst2, b2, W3, a_src3, a_dst3, b3):
    sd1 = _pad_edges(edge_index[0], edge_index[1])
    sd2 = _pad_edges(edge_index_2[0], edge_index_2[1])
    zrs = jnp.zeros((RPT, AUG), _f32)

    haug1, als1, ald1, mxs1, mxd1 = _tc_pre1(x, W1, a_src1, a_dst1)
    haug1_f = haug1.reshape(HEADS * N, AUG)
    als1_f = jnp.pad(als1.T, ((0, 0), (0, NPAD - N))).reshape(-1)
    ald1_f = jnp.pad(ald1.T, ((0, 0), (0, NPAD - N))).reshape(-1)

    acc1 = _sc_l1(haug1_f, sd1, als1_f, ald1_f,
                  mxs1.reshape(-1), mxd1.reshape(-1), zrs)
    acc1 = acc1.reshape(HEADS, NPAD, AUG)

    haug2, haug3, al23, mx23 = _tc_mid(
        acc1, b1.reshape(1, -1), W2, a_src2, a_dst2, W3, a_src3, a_dst3)
    haug23_f = jnp.concatenate([haug2, haug3], axis=0)
    al23_f = jnp.pad(al23.T, ((0, 0), (0, NPAD - N))).reshape(-1)
    sd23 = jnp.concatenate([sd1, sd2])

    acc23 = _sc_l23(haug23_f, sd23, al23_f, mx23.reshape(-1), zrs)

    logits, logits2, preds = _tc_post(acc23[:NPAD], acc23[NPAD:],
                                      b2.reshape(1, -1), b3.reshape(1, -1))
    return (logits, logits2, preds.reshape(-1))


# speedup vs baseline: 29.6940x; 1.0020x over previous
import functools

import jax
import jax.numpy as jnp
from jax import lax
from jax.experimental import pallas as pl
from jax.experimental.pallas import tpu as pltpu
from jax.experimental.pallas import tpu_sc as plsc

N = 10000
E = 320000
EP = E + N
F_IN = 128
HID = 64
HEADS = 8
OUT = 64
AUG = 128

NTILES = 16
B = 64
PER_TILE = 20736
EPAD = PER_TILE * NTILES
NPAD = 10240
RPT = NPAD // NTILES
NB = 10
BN = N // NB

_f32 = jnp.float32
_i32 = jnp.int32


def _tc_pre1_body(x_ref, w_ref, asr_ref, adr_ref,
                  haug_ref, als_ref, ald_ref, mxs_ref, mxd_ref):
    i = pl.program_id(0)

    @pl.when(i == 0)
    def _():
        mxs_ref[...] = jnp.full((HEADS, 128), -1e30, _f32)
        mxd_ref[...] = jnp.full((HEADS, 128), -1e30, _f32)

    h = jnp.dot(x_ref[...], w_ref[...], preferred_element_type=_f32)
    ones = jnp.ones((BN, 1), _f32)
    zpad = jnp.zeros((BN, AUG - HID - 1), _f32)
    a_s_all, a_d_all = [], []
    for hd in range(HEADS):
        hh = h[:, hd * HID:(hd + 1) * HID]
        a_s = jnp.sum(hh * asr_ref[hd, :][None, :], axis=1)
        a_d = jnp.sum(hh * adr_ref[hd, :][None, :], axis=1)
        a_s_all.append(a_s)
        a_d_all.append(a_d)
        mxs_ref[hd, :] = jnp.maximum(mxs_ref[hd, :], jnp.max(a_s))
        mxd_ref[hd, :] = jnp.maximum(mxd_ref[hd, :], jnp.max(a_d))
        haug_ref[hd, :, :] = jnp.concatenate([hh, ones, zpad], axis=1)
    als_ref[...] = jnp.stack(a_s_all, axis=1)
    ald_ref[...] = jnp.stack(a_d_all, axis=1)


def _tc_pre1(x, W1, a_src1, a_dst1):
    return pl.pallas_call(
        _tc_pre1_body,
        grid=(NB,),
        in_specs=[
            pl.BlockSpec((BN, F_IN), lambda i: (i, 0)),
            pl.BlockSpec((F_IN, HEADS * HID), lambda i: (0, 0)),
            pl.BlockSpec((HEADS, HID), lambda i: (0, 0)),
            pl.BlockSpec((HEADS, HID), lambda i: (0, 0)),
        ],
        out_specs=[
            pl.BlockSpec((HEADS, BN, AUG), lambda i: (0, i, 0)),
            pl.BlockSpec((BN, HEADS), lambda i: (i, 0)),
            pl.BlockSpec((BN, HEADS), lambda i: (i, 0)),
            pl.BlockSpec((HEADS, 128), lambda i: (0, 0)),
            pl.BlockSpec((HEADS, 128), lambda i: (0, 0)),
        ],
        out_shape=[
            jax.ShapeDtypeStruct((HEADS, N, AUG), _f32),
            jax.ShapeDtypeStruct((N, HEADS), _f32),
            jax.ShapeDtypeStruct((N, HEADS), _f32),
            jax.ShapeDtypeStruct((HEADS, 128), _f32),
            jax.ShapeDtypeStruct((HEADS, 128), _f32),
        ],
    )(x, W1, a_src1, a_dst1)


NCH = PER_TILE // B
NSC = (HID + 16) // 16


def _sc_chunk_loop(haug, sd, acc, tab_s, tab_d, sdbuf, dvec, gidx,
                   rows, eev, isem, gsem, ssem, bnd_v, cbase, goff):
    def idx_issue(c, s):
        off = (cbase + c) * (2 * B)
        pltpu.async_copy(sd.at[pl.ds(off, 2 * B)], sdbuf[s], isem[s])

    def prep(c, s):
        pltpu.make_async_copy(
            sd.at[pl.ds((cbase + c) * (2 * B), 2 * B)],
            sdbuf[s], isem[s]).wait()

        def sub16(k, _):
            o = k * 16
            sid = sdbuf[s][pl.ds(o, 16)]
            did = sdbuf[s][pl.ds(B + o, 16)]
            gidx[s][pl.ds(o, 16)] = sid + goff
            dvec[s][pl.ds(o, 16)] = did
            e = (plsc.load_gather(tab_s, [sid])
                 + plsc.load_gather(tab_d, [did]))
            e = jnp.where(e > 0, e, 0.2 * e)
            eev[s][pl.ds(o, 16)] = jnp.exp(e - bnd_v[...])
            return 0

        lax.fori_loop(0, B // 16, sub16, 0, unroll=4)
        pltpu.async_copy(haug.at[gidx[s]], rows[s], gsem[s])

    def drain(s):
        pltpu.make_async_copy(rows[s], acc.at[dvec[s]], ssem[s]).wait()

    def finish(s):
        pltpu.make_async_copy(haug.at[gidx[s]], rows[s], gsem[s]).wait()

        def scale(j, _):
            ej = plsc.load_gather(
                eev[s], [jnp.broadcast_to(j, (16,)).astype(_i32)])
            for q in range(NSC):
                rows[s][j, pl.ds(q * 16, 16)] = \
                    rows[s][j, pl.ds(q * 16, 16)] * ej
            return 0

        lax.fori_loop(0, B, scale, 0, unroll=8)
        pltpu.async_copy(rows[s], acc.at[dvec[s]], ssem[s], add=True)

    idx_issue(0, 0)
    idx_issue(1, 1)
    idx_issue(2, 2)
    prep(0, 0)
    prep(1, 1)

    def outer(g, _):
        for k in range(3):
            c = 3 * g + k
            s0 = k
            s2 = (k + 2) % 3

            @pl.when(c + 3 < NCH)
            def _():
                idx_issue(c + 3, s0)

            @pl.when(c + 2 < NCH)
            def _():
                @pl.when(c >= 1)
                def _():
                    drain(s2)
                prep(c + 2, s2)

            finish(s0)
        return 0

    lax.fori_loop(0, NCH // 3, outer, 0)
    drain(0)
    drain(1)
    drain(2)


def _sc_l1_body(haug, sd, alps, alpd, mxs, mxd, zrs, out,
                tab_s, tab_d, sdbuf0, sdbuf1, sdbuf2,
                dvec0, dvec1, dvec2, gidx0, gidx1, gidx2,
                rows0, rows1, rows2, eev0, eev1, eev2,
                m1v, m2v, bnd_v, acc, isem0, isem1, isem2,
                gsem0, gsem1, gsem2, ssem0, ssem1, ssem2):
    core = lax.axis_index("c")
    sub = lax.axis_index("s")
    cbase = sub * NCH
    rbase = sub * RPT
    for hl in range(HEADS // 2):
        head = core * (HEADS // 2) + hl
        pltpu.sync_copy(zrs, acc.at[pl.ds(rbase, RPT)])
        pltpu.sync_copy(alps.at[pl.ds(head * NPAD, NPAD)], tab_s)
        pltpu.sync_copy(alpd.at[pl.ds(head * NPAD, NPAD)], tab_d)
        pltpu.sync_copy(mxs.at[pl.ds(head * 128, 16)], m1v)
        pltpu.sync_copy(mxd.at[pl.ds(head * 128, 16)], m2v)
        b = m1v[...] + m2v[...]
        bnd_v[...] = jnp.where(b > 0, b, 0.2 * b)
        plsc.subcore_barrier()
        _sc_chunk_loop(haug, sd, acc, tab_s, tab_d,
                       (sdbuf0, sdbuf1, sdbuf2), (dvec0, dvec1, dvec2),
                       (gidx0, gidx1, gidx2), (rows0, rows1, rows2),
                       (eev0, eev1, eev2), (isem0, isem1, isem2),
                       (gsem0, gsem1, gsem2), (ssem0, ssem1, ssem2),
                       bnd_v, cbase, head * N)
        plsc.subcore_barrier()
        pltpu.sync_copy(acc.at[pl.ds(rbase, RPT)],
                        out.at[pl.ds(head * NPAD + rbase, RPT)])
        plsc.subcore_barrier()


def _sc_l1(haug_f, sd, alps_f, alpd_f, mxs_f, mxd_f, zrs):
    mesh = plsc.VectorSubcoreMesh(core_axis_name="c", subcore_axis_name="s")
    return pl.kernel(
        _sc_l1_body,
        out_type=jax.ShapeDtypeStruct((HEADS * NPAD, AUG), _f32),
        mesh=mesh,
        compiler_params=pltpu.CompilerParams(needs_layout_passes=False),
        scratch_types=_sc_scratch(),
    )(haug_f, sd, alps_f, alpd_f, mxs_f, mxd_f, zrs)


def _sc_scratch():
    ring = lambda t: [t, t, t]
    return (
        [pltpu.VMEM((NPAD,), _f32),
         pltpu.VMEM((NPAD,), _f32)]
        + ring(pltpu.VMEM((2 * B,), _i32))
        + ring(pltpu.VMEM((B,), _i32))
        + ring(pltpu.VMEM((B,), _i32))
        + ring(pltpu.VMEM((B, AUG), _f32))
        + ring(pltpu.VMEM((B,), _f32))
        + [pltpu.VMEM((16,), _f32),
           pltpu.VMEM((16,), _f32),
           pltpu.VMEM((16,), _f32),
           pltpu.VMEM_SHARED((NPAD, AUG), _f32)]
        + [pltpu.SemaphoreType.DMA] * 9
    )


def _sc_l23_body(haug, sd, alps, mxs, zrs, out,
                 tab_s, tab_d, sdbuf0, sdbuf1, sdbuf2,
                 dvec0, dvec1, dvec2, gidx0, gidx1, gidx2,
                 rows0, rows1, rows2, eev0, eev1, eev2,
                 m1v, m2v, bnd_v, acc, isem0, isem1, isem2,
                 gsem0, gsem1, gsem2, ssem0, ssem1, ssem2):
    core = lax.axis_index("c")
    sub = lax.axis_index("s")
    cbase = core * (EPAD // B) + sub * NCH
    rbase = sub * RPT
    row_s = 2 * core
    row_d = 2 * core + 1
    pltpu.sync_copy(zrs, acc.at[pl.ds(rbase, RPT)])
    pltpu.sync_copy(alps.at[pl.ds(row_s * NPAD, NPAD)], tab_s)
    pltpu.sync_copy(alps.at[pl.ds(row_d * NPAD, NPAD)], tab_d)
    pltpu.sync_copy(mxs.at[pl.ds(row_s * 128, 16)], m1v)
    pltpu.sync_copy(mxs.at[pl.ds(row_d * 128, 16)], m2v)
    b = m1v[...] + m2v[...]
    bnd_v[...] = jnp.where(b > 0, b, 0.2 * b)
    plsc.subcore_barrier()
    _sc_chunk_loop(haug, sd, acc, tab_s, tab_d,
                   (sdbuf0, sdbuf1, sdbuf2), (dvec0, dvec1, dvec2),
                   (gidx0, gidx1, gidx2), (rows0, rows1, rows2),
                   (eev0, eev1, eev2), (isem0, isem1, isem2),
                   (gsem0, gsem1, gsem2), (ssem0, ssem1, ssem2),
                   bnd_v, cbase, core * N)
    plsc.subcore_barrier()
    pltpu.sync_copy(acc.at[pl.ds(rbase, RPT)],
                    out.at[pl.ds(core * NPAD + rbase, RPT)])


def _sc_l23(haug_f, sd2, alps_f, mxs_f, zrs):
    mesh = plsc.VectorSubcoreMesh(core_axis_name="c", subcore_axis_name="s")
    return pl.kernel(
        _sc_l23_body,
        out_type=jax.ShapeDtypeStruct((2 * NPAD, AUG), _f32),
        mesh=mesh,
        compiler_params=pltpu.CompilerParams(needs_layout_passes=False),
        scratch_types=_sc_scratch(),
    )(haug_f, sd2, alps_f, mxs_f, zrs)


def _tc_mid_body(acc_ref, b1_ref, w2_ref, as2_ref, ad2_ref,
                 w3_ref, as3_ref, ad3_ref,
                 haug2_ref, haug3_ref, al_ref, mx_ref):
    i = pl.program_id(0)

    @pl.when(i == 0)
    def _():
        mx_ref[...] = jnp.full((HEADS, 128), -1e30, _f32)

    cols = []
    for hd in range(HEADS):
        num = acc_ref[hd, :, 0:HID]
        den = acc_ref[hd, :, HID:HID + 1]
        cols.append(num / (den + 1e-16))
    emb = jnp.concatenate(cols, axis=1) + b1_ref[0, :][None, :]
    emb = jnp.where(emb > 0, emb, jnp.exp(jnp.minimum(emb, 0.0)) - 1.0)

    ones = jnp.ones((BN, 1), _f32)
    zpad = jnp.zeros((BN, AUG - OUT - 1), _f32)

    h2 = jnp.dot(emb, w2_ref[...], preferred_element_type=_f32)
    a_s2 = jnp.sum(h2 * as2_ref[0, :][None, :], axis=1)
    a_d2 = jnp.sum(h2 * ad2_ref[0, :][None, :], axis=1)
    haug2_ref[...] = jnp.concatenate([h2, ones, zpad], axis=1)

    h3 = jnp.dot(emb, w3_ref[...], preferred_element_type=_f32)
    a_s3 = jnp.sum(h3 * as3_ref[0, :][None, :], axis=1)
    a_d3 = jnp.sum(h3 * ad3_ref[0, :][None, :], axis=1)
    haug3_ref[...] = jnp.concatenate([h3, ones, zpad], axis=1)

    zcol = jnp.zeros((BN,), _f32)
    al_ref[...] = jnp.stack(
        [a_s2, a_d2, a_s3, a_d3] + [zcol] * (HEADS - 4), axis=1)
    mx_ref[0, :] = jnp.maximum(mx_ref[0, :], jnp.max(a_s2))
    mx_ref[1, :] = jnp.maximum(mx_ref[1, :], jnp.max(a_d2))
    mx_ref[2, :] = jnp.maximum(mx_ref[2, :], jnp.max(a_s3))
    mx_ref[3, :] = jnp.maximum(mx_ref[3, :], jnp.max(a_d3))


def _tc_mid(acc1, b1, W2, a_src2, a_dst2, W3, a_src3, a_dst3):
    return pl.pallas_call(
        _tc_mid_body,
        grid=(NB,),
        in_specs=[
            pl.BlockSpec((HEADS, BN, AUG), lambda i: (0, i, 0)),
            pl.BlockSpec((1, HEADS * HID), lambda i: (0, 0)),
            pl.BlockSpec((HEADS * HID, OUT), lambda i: (0, 0)),
            pl.BlockSpec((1, OUT), lambda i: (0, 0)),
            pl.BlockSpec((1, OUT), lambda i: (0, 0)),
            pl.BlockSpec((HEADS * HID, OUT), lambda i: (0, 0)),
            pl.BlockSpec((1, OUT), lambda i: (0, 0)),
            pl.BlockSpec((1, OUT), lambda i: (0, 0)),
        ],
        out_specs=[
            pl.BlockSpec((BN, AUG), lambda i: (i, 0)),
            pl.BlockSpec((BN, AUG), lambda i: (i, 0)),
            pl.BlockSpec((BN, HEADS), lambda i: (i, 0)),
            pl.BlockSpec((HEADS, 128), lambda i: (0, 0)),
        ],
        out_shape=[
            jax.ShapeDtypeStruct((N, AUG), _f32),
            jax.ShapeDtypeStruct((N, AUG), _f32),
            jax.ShapeDtypeStruct((N, HEADS), _f32),
            jax.ShapeDtypeStruct((HEADS, 128), _f32),
        ],
    )(acc1, b1, W2, a_src2, a_dst2, W3, a_src3, a_dst3)


def _tc_post_body(acc2_ref, acc3_ref, b2_ref, b3_ref,
                  lg1_ref, lg2_ref, prd_ref):
    x1 = acc2_ref[:, 0:OUT] / (acc2_ref[:, HID:HID + 1] + 1e-16) \
        + b2_ref[0, :][None, :]
    x2 = acc3_ref[:, 0:OUT] / (acc3_ref[:, HID:HID + 1] + 1e-16) \
        + b3_ref[0, :][None, :]
    m1 = jnp.max(x1, axis=1, keepdims=True)
    p1 = jnp.exp(x1 - m1)
    lg1_ref[...] = p1 / jnp.sum(p1, axis=1, keepdims=True)
    m2 = jnp.max(x2, axis=1, keepdims=True)
    p2 = jnp.exp(x2 - m2)
    lg2_ref[...] = p2 / jnp.sum(p2, axis=1, keepdims=True)
    ii = lax.broadcasted_iota(_i32, (BN, OUT), 1)
    cand = jnp.where(x1 == m1, ii, OUT)
    prd_ref[0, 0, :] = jnp.min(cand, axis=1)


def _tc_post(acc2, acc3, b2, b3):
    return pl.pallas_call(
        _tc_post_body,
        grid=(NB,),
        in_specs=[
            pl.BlockSpec((BN, AUG), lambda i: (i, 0)),
            pl.BlockSpec((BN, AUG), lambda i: (i, 0)),
            pl.BlockSpec((1, OUT), lambda i: (0, 0)),
            pl.BlockSpec((1, OUT), lambda i: (0, 0)),
        ],
        out_specs=[
            pl.BlockSpec((BN, OUT), lambda i: (i, 0)),
            pl.BlockSpec((BN, OUT), lambda i: (i, 0)),
            pl.BlockSpec((1, 1, BN), lambda i: (i, 0, 0)),
        ],
        out_shape=[
            jax.ShapeDtypeStruct((N, OUT), _f32),
            jax.ShapeDtypeStruct((N, OUT), _f32),
            jax.ShapeDtypeStruct((NB, 1, BN), _i32),
        ],
    )(acc2, acc3, b2, b3)


def _pad_edges(src, dst):
    loop = jnp.arange(N, dtype=_i32)
    npad = EPAD - EP
    s = jnp.concatenate([src.astype(_i32), loop,
                         jnp.zeros((npad,), _i32)])
    d = jnp.concatenate([dst.astype(_i32), loop,
                         jnp.full((npad,), N, _i32)])
    return jnp.stack([s.reshape(-1, B), d.reshape(-1, B)],
                     axis=1).reshape(-1)


def kernel(x, edge_index, edge_index_2, W1, a_src1, a_dst1, b1,
           W2, a_src2, a_dst2, b2, W3, a_src3, a_dst3, b3):
    sd1 = _pad_edges(edge_index[0], edge_index[1])
    sd2 = _pad_edges(edge_index_2[0], edge_index_2[1])
    zrs = jnp.zeros((RPT, AUG), _f32)

    haug1, als1, ald1, mxs1, mxd1 = _tc_pre1(x, W1, a_src1, a_dst1)
    haug1_f = haug1.reshape(HEADS * N, AUG)
    als1_f = jnp.pad(als1.T, ((0, 0), (0, NPAD - N))).reshape(-1)
    ald1_f = jnp.pad(ald1.T, ((0, 0), (0, NPAD - N))).reshape(-1)

    acc1 = _sc_l1(haug1_f, sd1, als1_f, ald1_f,
                  mxs1.reshape(-1), mxd1.reshape(-1), zrs)
    acc1 = acc1.reshape(HEADS, NPAD, AUG)

    haug2, haug3, al23, mx23 = _tc_mid(
        acc1, b1.reshape(1, -1), W2, a_src2, a_dst2, W3, a_src3, a_dst3)
    haug23_f = jnp.concatenate([haug2, haug3], axis=0)
    al23_f = jnp.pad(al23.T, ((0, 0), (0, NPAD - N))).reshape(-1)
    sd23 = jnp.concatenate([sd1, sd2])

    acc23 = _sc_l23(haug23_f, sd23, al23_f, mx23.reshape(-1), zrs)

    logits, logits2, preds = _tc_post(acc23[:NPAD], acc23[NPAD:],
                                      b2.reshape(1, -1), b3.reshape(1, -1))
    return (logits, logits2, preds.reshape(-1))


# speedup vs baseline: 33.3526x; 1.1232x over previous
import functools

import jax
import jax.numpy as jnp
from jax import lax
from jax.experimental import pallas as pl
from jax.experimental.pallas import tpu as pltpu
from jax.experimental.pallas import tpu_sc as plsc

N = 10000
E = 320000
EP = E + N
F_IN = 128
HID = 64
HEADS = 8
OUT = 64
AUG = 80

NTILES = 16
B = 64
PER_TILE = 20736
EPAD = PER_TILE * NTILES
NPAD = 10240
RPT = NPAD // NTILES
NB = 10
BN = N // NB

_f32 = jnp.float32
_i32 = jnp.int32


def _tc_pre1_body(x_ref, w_ref, asr_ref, adr_ref,
                  haug_ref, als_ref, ald_ref, mxs_ref, mxd_ref):
    i = pl.program_id(0)

    @pl.when(i == 0)
    def _():
        mxs_ref[...] = jnp.full((HEADS, 128), -1e30, _f32)
        mxd_ref[...] = jnp.full((HEADS, 128), -1e30, _f32)

    h = jnp.dot(x_ref[...], w_ref[...], preferred_element_type=_f32)
    ones = jnp.ones((BN, 1), _f32)
    zpad = jnp.zeros((BN, AUG - HID - 1), _f32)
    a_s_all, a_d_all = [], []
    for hd in range(HEADS):
        hh = h[:, hd * HID:(hd + 1) * HID]
        a_s = jnp.sum(hh * asr_ref[hd, :][None, :], axis=1)
        a_d = jnp.sum(hh * adr_ref[hd, :][None, :], axis=1)
        a_s_all.append(a_s)
        a_d_all.append(a_d)
        mxs_ref[hd, :] = jnp.maximum(mxs_ref[hd, :], jnp.max(a_s))
        mxd_ref[hd, :] = jnp.maximum(mxd_ref[hd, :], jnp.max(a_d))
        haug_ref[hd, :, :] = jnp.concatenate([hh, ones, zpad], axis=1)
    als_ref[...] = jnp.stack(a_s_all, axis=1)
    ald_ref[...] = jnp.stack(a_d_all, axis=1)


def _tc_pre1(x, W1, a_src1, a_dst1):
    return pl.pallas_call(
        _tc_pre1_body,
        grid=(NB,),
        in_specs=[
            pl.BlockSpec((BN, F_IN), lambda i: (i, 0)),
            pl.BlockSpec((F_IN, HEADS * HID), lambda i: (0, 0)),
            pl.BlockSpec((HEADS, HID), lambda i: (0, 0)),
            pl.BlockSpec((HEADS, HID), lambda i: (0, 0)),
        ],
        out_specs=[
            pl.BlockSpec((HEADS, BN, AUG), lambda i: (0, i, 0)),
            pl.BlockSpec((BN, HEADS), lambda i: (i, 0)),
            pl.BlockSpec((BN, HEADS), lambda i: (i, 0)),
            pl.BlockSpec((HEADS, 128), lambda i: (0, 0)),
            pl.BlockSpec((HEADS, 128), lambda i: (0, 0)),
        ],
        out_shape=[
            jax.ShapeDtypeStruct((HEADS, N, AUG), _f32),
            jax.ShapeDtypeStruct((N, HEADS), _f32),
            jax.ShapeDtypeStruct((N, HEADS), _f32),
            jax.ShapeDtypeStruct((HEADS, 128), _f32),
            jax.ShapeDtypeStruct((HEADS, 128), _f32),
        ],
    )(x, W1, a_src1, a_dst1)


NCH = PER_TILE // B
NSC = (HID + 16) // 16


def _sc_chunk_loop(haug, sd, acc, tab_s, tab_d, sdbuf, dvec, gidx,
                   rows, eev, isem, gsem, ssem, bnd_v, cbase, goff):
    def idx_issue(c, s):
        off = (cbase + c) * (2 * B)
        pltpu.async_copy(sd.at[pl.ds(off, 2 * B)], sdbuf[s], isem[s])

    def prep(c, s):
        pltpu.make_async_copy(
            sd.at[pl.ds((cbase + c) * (2 * B), 2 * B)],
            sdbuf[s], isem[s]).wait()

        def sub16(k, _):
            o = k * 16
            sid = sdbuf[s][pl.ds(o, 16)]
            did = sdbuf[s][pl.ds(B + o, 16)]
            gidx[s][pl.ds(o, 16)] = sid + goff
            dvec[s][pl.ds(o, 16)] = did
            e = (plsc.load_gather(tab_s, [sid])
                 + plsc.load_gather(tab_d, [did]))
            e = jnp.where(e > 0, e, 0.2 * e)
            eev[s][pl.ds(o, 16)] = jnp.exp(e - bnd_v[...])
            return 0

        lax.fori_loop(0, B // 16, sub16, 0, unroll=4)
        pltpu.async_copy(haug.at[gidx[s]], rows[s], gsem[s])

    def drain(s):
        pltpu.make_async_copy(rows[s], acc.at[dvec[s]], ssem[s]).wait()

    def finish(s):
        pltpu.make_async_copy(haug.at[gidx[s]], rows[s], gsem[s]).wait()

        def scale(j, _):
            ej = plsc.load_gather(
                eev[s], [jnp.broadcast_to(j, (16,)).astype(_i32)])
            for q in range(NSC):
                rows[s][j, pl.ds(q * 16, 16)] = \
                    rows[s][j, pl.ds(q * 16, 16)] * ej
            return 0

        lax.fori_loop(0, B, scale, 0, unroll=8)
        pltpu.async_copy(rows[s], acc.at[dvec[s]], ssem[s], add=True)

    idx_issue(0, 0)
    idx_issue(1, 1)
    idx_issue(2, 2)
    prep(0, 0)
    prep(1, 1)

    def outer(g, _):
        for k in range(3):
            c = 3 * g + k
            s0 = k
            s2 = (k + 2) % 3

            @pl.when(c + 3 < NCH)
            def _():
                idx_issue(c + 3, s0)

            @pl.when(c + 2 < NCH)
            def _():
                @pl.when(c >= 1)
                def _():
                    drain(s2)
                prep(c + 2, s2)

            finish(s0)
        return 0

    lax.fori_loop(0, NCH // 3, outer, 0)
    drain(0)
    drain(1)
    drain(2)


def _sc_l1_body(haug, sd, alps, alpd, mxs, mxd, zrs, out,
                tab_s, tab_d, sdbuf0, sdbuf1, sdbuf2,
                dvec0, dvec1, dvec2, gidx0, gidx1, gidx2,
                rows0, rows1, rows2, eev0, eev1, eev2,
                m1v, m2v, bnd_v, acc, isem0, isem1, isem2,
                gsem0, gsem1, gsem2, ssem0, ssem1, ssem2):
    core = lax.axis_index("c")
    sub = lax.axis_index("s")
    cbase = sub * NCH
    rbase = sub * RPT
    for hl in range(HEADS // 2):
        head = core * (HEADS // 2) + hl
        pltpu.sync_copy(zrs, acc.at[pl.ds(rbase, RPT)])
        pltpu.sync_copy(alps.at[pl.ds(head * NPAD, NPAD)], tab_s)
        pltpu.sync_copy(alpd.at[pl.ds(head * NPAD, NPAD)], tab_d)
        pltpu.sync_copy(mxs.at[pl.ds(head * 128, 16)], m1v)
        pltpu.sync_copy(mxd.at[pl.ds(head * 128, 16)], m2v)
        b = m1v[...] + m2v[...]
        bnd_v[...] = jnp.where(b > 0, b, 0.2 * b)
        plsc.subcore_barrier()
        _sc_chunk_loop(haug, sd, acc, tab_s, tab_d,
                       (sdbuf0, sdbuf1, sdbuf2), (dvec0, dvec1, dvec2),
                       (gidx0, gidx1, gidx2), (rows0, rows1, rows2),
                       (eev0, eev1, eev2), (isem0, isem1, isem2),
                       (gsem0, gsem1, gsem2), (ssem0, ssem1, ssem2),
                       bnd_v, cbase, head * N)
        plsc.subcore_barrier()
        pltpu.sync_copy(acc.at[pl.ds(rbase, RPT)],
                        out.at[pl.ds(head * NPAD + rbase, RPT)])
        plsc.subcore_barrier()


def _sc_l1(haug_f, sd, alps_f, alpd_f, mxs_f, mxd_f, zrs):
    mesh = plsc.VectorSubcoreMesh(core_axis_name="c", subcore_axis_name="s")
    return pl.kernel(
        _sc_l1_body,
        out_type=jax.ShapeDtypeStruct((HEADS * NPAD, AUG), _f32),
        mesh=mesh,
        compiler_params=pltpu.CompilerParams(
            needs_layout_passes=False, use_tc_tiling_on_sc=False),
        scratch_types=_sc_scratch(),
    )(haug_f, sd, alps_f, alpd_f, mxs_f, mxd_f, zrs)


def _sc_scratch():
    ring = lambda t: [t, t, t]
    return (
        [pltpu.VMEM((NPAD,), _f32),
         pltpu.VMEM((NPAD,), _f32)]
        + ring(pltpu.VMEM((2 * B,), _i32))
        + ring(pltpu.VMEM((B,), _i32))
        + ring(pltpu.VMEM((B,), _i32))
        + ring(pltpu.VMEM((B, AUG), _f32))
        + ring(pltpu.VMEM((B,), _f32))
        + [pltpu.VMEM((16,), _f32),
           pltpu.VMEM((16,), _f32),
           pltpu.VMEM((16,), _f32),
           pltpu.VMEM_SHARED((NPAD, AUG), _f32)]
        + [pltpu.SemaphoreType.DMA] * 9
    )


def _sc_l23_body(haug, sd, alps, mxs, zrs, out,
                 tab_s, tab_d, sdbuf0, sdbuf1, sdbuf2,
                 dvec0, dvec1, dvec2, gidx0, gidx1, gidx2,
                 rows0, rows1, rows2, eev0, eev1, eev2,
                 m1v, m2v, bnd_v, acc, isem0, isem1, isem2,
                 gsem0, gsem1, gsem2, ssem0, ssem1, ssem2):
    core = lax.axis_index("c")
    sub = lax.axis_index("s")
    cbase = core * (EPAD // B) + sub * NCH
    rbase = sub * RPT
    row_s = 2 * core
    row_d = 2 * core + 1
    pltpu.sync_copy(zrs, acc.at[pl.ds(rbase, RPT)])
    pltpu.sync_copy(alps.at[pl.ds(row_s * NPAD, NPAD)], tab_s)
    pltpu.sync_copy(alps.at[pl.ds(row_d * NPAD, NPAD)], tab_d)
    pltpu.sync_copy(mxs.at[pl.ds(row_s * 128, 16)], m1v)
    pltpu.sync_copy(mxs.at[pl.ds(row_d * 128, 16)], m2v)
    b = m1v[...] + m2v[...]
    bnd_v[...] = jnp.where(b > 0, b, 0.2 * b)
    plsc.subcore_barrier()
    _sc_chunk_loop(haug, sd, acc, tab_s, tab_d,
                   (sdbuf0, sdbuf1, sdbuf2), (dvec0, dvec1, dvec2),
                   (gidx0, gidx1, gidx2), (rows0, rows1, rows2),
                   (eev0, eev1, eev2), (isem0, isem1, isem2),
                   (gsem0, gsem1, gsem2), (ssem0, ssem1, ssem2),
                   bnd_v, cbase, core * N)
    plsc.subcore_barrier()
    pltpu.sync_copy(acc.at[pl.ds(rbase, RPT)],
                    out.at[pl.ds(core * NPAD + rbase, RPT)])


def _sc_l23(haug_f, sd2, alps_f, mxs_f, zrs):
    mesh = plsc.VectorSubcoreMesh(core_axis_name="c", subcore_axis_name="s")
    return pl.kernel(
        _sc_l23_body,
        out_type=jax.ShapeDtypeStruct((2 * NPAD, AUG), _f32),
        mesh=mesh,
        compiler_params=pltpu.CompilerParams(
            needs_layout_passes=False, use_tc_tiling_on_sc=False),
        scratch_types=_sc_scratch(),
    )(haug_f, sd2, alps_f, mxs_f, zrs)


def _tc_mid_body(acc_ref, b1_ref, w2_ref, as2_ref, ad2_ref,
                 w3_ref, as3_ref, ad3_ref,
                 haug2_ref, haug3_ref, al_ref, mx_ref):
    i = pl.program_id(0)

    @pl.when(i == 0)
    def _():
        mx_ref[...] = jnp.full((HEADS, 128), -1e30, _f32)

    cols = []
    for hd in range(HEADS):
        num = acc_ref[hd, :, 0:HID]
        den = acc_ref[hd, :, HID:HID + 1]
        cols.append(num / (den + 1e-16))
    emb = jnp.concatenate(cols, axis=1) + b1_ref[0, :][None, :]
    emb = jnp.where(emb > 0, emb, jnp.exp(jnp.minimum(emb, 0.0)) - 1.0)

    ones = jnp.ones((BN, 1), _f32)
    zpad = jnp.zeros((BN, AUG - OUT - 1), _f32)

    h2 = jnp.dot(emb, w2_ref[...], preferred_element_type=_f32)
    a_s2 = jnp.sum(h2 * as2_ref[0, :][None, :], axis=1)
    a_d2 = jnp.sum(h2 * ad2_ref[0, :][None, :], axis=1)
    haug2_ref[...] = jnp.concatenate([h2, ones, zpad], axis=1)

    h3 = jnp.dot(emb, w3_ref[...], preferred_element_type=_f32)
    a_s3 = jnp.sum(h3 * as3_ref[0, :][None, :], axis=1)
    a_d3 = jnp.sum(h3 * ad3_ref[0, :][None, :], axis=1)
    haug3_ref[...] = jnp.concatenate([h3, ones, zpad], axis=1)

    zcol = jnp.zeros((BN,), _f32)
    al_ref[...] = jnp.stack(
        [a_s2, a_d2, a_s3, a_d3] + [zcol] * (HEADS - 4), axis=1)
    mx_ref[0, :] = jnp.maximum(mx_ref[0, :], jnp.max(a_s2))
    mx_ref[1, :] = jnp.maximum(mx_ref[1, :], jnp.max(a_d2))
    mx_ref[2, :] = jnp.maximum(mx_ref[2, :], jnp.max(a_s3))
    mx_ref[3, :] = jnp.maximum(mx_ref[3, :], jnp.max(a_d3))


def _tc_mid(acc1, b1, W2, a_src2, a_dst2, W3, a_src3, a_dst3):
    return pl.pallas_call(
        _tc_mid_body,
        grid=(NB,),
        in_specs=[
            pl.BlockSpec((HEADS, BN, AUG), lambda i: (0, i, 0)),
            pl.BlockSpec((1, HEADS * HID), lambda i: (0, 0)),
            pl.BlockSpec((HEADS * HID, OUT), lambda i: (0, 0)),
            pl.BlockSpec((1, OUT), lambda i: (0, 0)),
            pl.BlockSpec((1, OUT), lambda i: (0, 0)),
            pl.BlockSpec((HEADS * HID, OUT), lambda i: (0, 0)),
            pl.BlockSpec((1, OUT), lambda i: (0, 0)),
            pl.BlockSpec((1, OUT), lambda i: (0, 0)),
        ],
        out_specs=[
            pl.BlockSpec((BN, AUG), lambda i: (i, 0)),
            pl.BlockSpec((BN, AUG), lambda i: (i, 0)),
            pl.BlockSpec((BN, HEADS), lambda i: (i, 0)),
            pl.BlockSpec((HEADS, 128), lambda i: (0, 0)),
        ],
        out_shape=[
            jax.ShapeDtypeStruct((N, AUG), _f32),
            jax.ShapeDtypeStruct((N, AUG), _f32),
            jax.ShapeDtypeStruct((N, HEADS), _f32),
            jax.ShapeDtypeStruct((HEADS, 128), _f32),
        ],
    )(acc1, b1, W2, a_src2, a_dst2, W3, a_src3, a_dst3)


def _tc_post_body(acc2_ref, acc3_ref, b2_ref, b3_ref,
                  lg1_ref, lg2_ref, prd_ref):
    x1 = acc2_ref[:, 0:OUT] / (acc2_ref[:, HID:HID + 1] + 1e-16) \
        + b2_ref[0, :][None, :]
    x2 = acc3_ref[:, 0:OUT] / (acc3_ref[:, HID:HID + 1] + 1e-16) \
        + b3_ref[0, :][None, :]
    m1 = jnp.max(x1, axis=1, keepdims=True)
    p1 = jnp.exp(x1 - m1)
    lg1_ref[...] = p1 / jnp.sum(p1, axis=1, keepdims=True)
    m2 = jnp.max(x2, axis=1, keepdims=True)
    p2 = jnp.exp(x2 - m2)
    lg2_ref[...] = p2 / jnp.sum(p2, axis=1, keepdims=True)
    ii = lax.broadcasted_iota(_i32, (BN, OUT), 1)
    cand = jnp.where(x1 == m1, ii, OUT)
    prd_ref[0, 0, :] = jnp.min(cand, axis=1)


def _tc_post(acc2, acc3, b2, b3):
    return pl.pallas_call(
        _tc_post_body,
        grid=(NB,),
        in_specs=[
            pl.BlockSpec((BN, AUG), lambda i: (i, 0)),
            pl.BlockSpec((BN, AUG), lambda i: (i, 0)),
            pl.BlockSpec((1, OUT), lambda i: (0, 0)),
            pl.BlockSpec((1, OUT), lambda i: (0, 0)),
        ],
        out_specs=[
            pl.BlockSpec((BN, OUT), lambda i: (i, 0)),
            pl.BlockSpec((BN, OUT), lambda i: (i, 0)),
            pl.BlockSpec((1, 1, BN), lambda i: (i, 0, 0)),
        ],
        out_shape=[
            jax.ShapeDtypeStruct((N, OUT), _f32),
            jax.ShapeDtypeStruct((N, OUT), _f32),
            jax.ShapeDtypeStruct((NB, 1, BN), _i32),
        ],
    )(acc2, acc3, b2, b3)


def _pad_edges(src, dst):
    loop = jnp.arange(N, dtype=_i32)
    npad = EPAD - EP
    s = jnp.concatenate([src.astype(_i32), loop,
                         jnp.zeros((npad,), _i32)])
    d = jnp.concatenate([dst.astype(_i32), loop,
                         jnp.full((npad,), N, _i32)])
    return jnp.stack([s.reshape(-1, B), d.reshape(-1, B)],
                     axis=1).reshape(-1)


def kernel(x, edge_index, edge_index_2, W1, a_src1, a_dst1, b1,
           W2, a_src2, a_dst2, b2, W3, a_src3, a_dst3, b3):
    sd1 = _pad_edges(edge_index[0], edge_index[1])
    sd2 = _pad_edges(edge_index_2[0], edge_index_2[1])
    zrs = jnp.zeros((RPT, AUG), _f32)

    haug1, als1, ald1, mxs1, mxd1 = _tc_pre1(x, W1, a_src1, a_dst1)
    haug1_f = haug1.reshape(HEADS * N, AUG)
    als1_f = jnp.pad(als1.T, ((0, 0), (0, NPAD - N))).reshape(-1)
    ald1_f = jnp.pad(ald1.T, ((0, 0), (0, NPAD - N))).reshape(-1)

    acc1 = _sc_l1(haug1_f, sd1, als1_f, ald1_f,
                  mxs1.reshape(-1), mxd1.reshape(-1), zrs)
    acc1 = acc1.reshape(HEADS, NPAD, AUG)

    haug2, haug3, al23, mx23 = _tc_mid(
        acc1, b1.reshape(1, -1), W2, a_src2, a_dst2, W3, a_src3, a_dst3)
    haug23_f = jnp.concatenate([haug2, haug3], axis=0)
    al23_f = jnp.pad(al23.T, ((0, 0), (0, NPAD - N))).reshape(-1)
    sd23 = jnp.concatenate([sd1, sd2])

    acc23 = _sc_l23(haug23_f, sd23, al23_f, mx23.reshape(-1), zrs)

    logits, logits2, preds = _tc_post(acc23[:NPAD], acc23[NPAD:],
                                      b2.reshape(1, -1), b3.reshape(1, -1))
    return (logits, logits2, preds.reshape(-1))
